# baseline (speedup 1.0000x reference)
import functools

import jax
import jax.numpy as jnp
import numpy as np
from jax import lax
from jax.experimental import pallas as pl
from jax.experimental.pallas import tpu as pltpu
from jax.experimental.pallas import tpu_sc as plsc

N = 10000
E = 160000
NE = 4
C = 128
RMAX = 5.0
NB = 8
P = 5
AVG = 16.0
C2M = 1.6792

NCORES = 2
NSUB = 16
EP = 163840
GCH = EP // (NCORES * NSUB)
MCH = EP // NSUB
NCHUNK = 128
NACC = 10240
DUMMY = N
NWB = NACC // NSUB

BE = 2048
BN = 2000

_I32 = jnp.int32
_F32 = jnp.float32


def _silu(x):
    return x * jax.nn.sigmoid(x)


def _full16(v):
    return jnp.full((16,), v, _I32)


def _geom_body(pos_hbm, send_hbm, recv_hbm, shift_hbm, vec_hbm,
               posb, sbuf, rbuf, shb, vb):
    c = lax.axis_index("c")
    s = lax.axis_index("s")
    wid = s * NCORES + c
    base = wid * GCH
    pltpu.sync_copy(pos_hbm, posb)
    pltpu.sync_copy(send_hbm.at[pl.ds(base, GCH)], sbuf)
    pltpu.sync_copy(recv_hbm.at[pl.ds(base, GCH)], rbuf)
    pltpu.sync_copy(shift_hbm.at[pl.ds(base * 3, GCH * 3)], shb)
    iota16 = lax.iota(_I32, 16)
    three = _full16(3)

    @pl.loop(0, GCH // 16)
    def _micro(m):
        off = pl.multiple_of(m * 16, 16)
        sidx = sbuf[pl.ds(off, 16)] * three
        ridx = rbuf[pl.ds(off, 16)] * three
        lidx = (jnp.full((16,), off, _I32) + iota16) * three
        for k in range(3):
            kc = _full16(k)
            p_s = plsc.load_gather(posb, [sidx + kc])
            p_r = plsc.load_gather(posb, [ridx + kc])
            sh = plsc.load_gather(shb, [lidx + kc])
            plsc.store_scatter(vb, [lidx + kc], p_r - p_s + sh)

    pltpu.sync_copy(vb, vec_hbm.at[pl.ds(base * 3, GCH * 3)])


def _geometry(positions, send_p, recv_p, shifts_p):
    mesh = plsc.VectorSubcoreMesh(core_axis_name="c", subcore_axis_name="s")
    return pl.kernel(
        _geom_body,
        out_type=jax.ShapeDtypeStruct((EP * 3,), _F32),
        mesh=mesh,
        compiler_params=pltpu.CompilerParams(needs_layout_passes=False),
        scratch_types=[
            pltpu.MemorySpace.VMEM((N * 3,), _F32),
            pltpu.MemorySpace.VMEM((GCH,), _I32),
            pltpu.MemorySpace.VMEM((GCH,), _I32),
            pltpu.MemorySpace.VMEM((GCH * 3,), _F32),
            pltpu.MemorySpace.VMEM((GCH * 3,), _F32),
        ],
    )(positions, send_p, recv_p, shifts_p)


def _msg_body(send_hbm, recv_hbm, hg0, hg1, hg2, hg3, rw0, rw1, rw2, rw3,
              sh_hbm, agg0, agg1, agg2, agg3,
              acc, zbuf, sidx, ridx, hbuf, rwbuf, shbuf, msgbuf, sem):
    c = lax.axis_index("c")
    s = lax.axis_index("s")
    hgs = (hg0, hg1, hg2, hg3)
    rws = (rw0, rw1, rw2, rw3)
    aggs = (agg0, agg1, agg2, agg3)
    zero16 = jnp.zeros((16,), _F32)
    c1 = _full16(1)
    c2 = _full16(2)
    c3 = _full16(3)

    @pl.loop(0, NCHUNK)
    def _z(i):
        for j in range(8):
            zbuf[i, 16 * j:16 * (j + 1)] = zero16

    for p in range(2):
        for z in range(NACC // (NSUB * NCHUNK)):
            pltpu.sync_copy(zbuf, acc.at[pl.ds((s * (NACC // (NSUB * NCHUNK)) + z) * NCHUNK, NCHUNK)])
        plsc.subcore_barrier()

        @pl.loop(0, MCH // NCHUNK)
        def _chunk(j):
            e0 = s * MCH + j * NCHUNK
            pltpu.sync_copy(send_hbm.at[pl.ds(e0, NCHUNK)], sidx)
            pltpu.sync_copy(recv_hbm.at[pl.ds(e0, NCHUNK)], ridx)
            pltpu.sync_copy(sh_hbm.at[pl.ds(e0 * 4, NCHUNK * 4)],
                            shbuf.at[pl.ds(0, NCHUNK * 4)])
            for cs in range(NCORES):
                g = NCORES * cs + p

                @pl.when(c == cs)
                def _():
                    pltpu.sync_copy(rws[g].at[pl.ds(e0, NCHUNK)], rwbuf)
                    pltpu.async_copy(hgs[g].at[sidx], hbuf, sem).wait()

            @pl.loop(0, NCHUNK)
            def _edge(i):
                ha = hbuf[i, 0:16]
                hb = hbuf[i, 16:32]
                r0a = rwbuf[i, 0:16]
                r0b = rwbuf[i, 16:32]
                r1a = rwbuf[i, 32:48]
                r1b = rwbuf[i, 48:64]
                sv = shbuf[pl.ds(i * 4, 16)]
                s1 = sv[c1]
                s2 = sv[c2]
                s3 = sv[c3]
                h1a = ha * r1a
                h1b = hb * r1b
                msgbuf[i, 0:16] = ha * r0a
                msgbuf[i, 16:32] = hb * r0b
                msgbuf[i, 32:48] = h1a * s1
                msgbuf[i, 48:64] = h1b * s1
                msgbuf[i, 64:80] = h1a * s2
                msgbuf[i, 80:96] = h1b * s2
                msgbuf[i, 96:112] = h1a * s3
                msgbuf[i, 112:128] = h1b * s3

            pltpu.sync_copy(msgbuf, acc.at[ridx], add=True)

        plsc.subcore_barrier()
        for cs in range(NCORES):
            g = NCORES * cs + p

            @pl.when(c == cs)
            def _():
                pltpu.sync_copy(acc.at[pl.ds(s * NWB, NWB)],
                                aggs[g].at[pl.ds(s * NWB, NWB)])
        plsc.subcore_barrier()


def _message(send_p, recv_p, hgs, rwgs, sh_p):
    mesh = plsc.VectorSubcoreMesh(core_axis_name="c", subcore_axis_name="s")
    return pl.kernel(
        _msg_body,
        out_type=[jax.ShapeDtypeStruct((NACC, C), _F32)] * 4,
        mesh=mesh,
        compiler_params=pltpu.CompilerParams(needs_layout_passes=False,
                                             use_tc_tiling_on_sc=False),
        scratch_types=[
            pltpu.MemorySpace.VMEM_SHARED((NACC, C), _F32),
            pltpu.MemorySpace.VMEM((NCHUNK, C), _F32),
            pltpu.MemorySpace.VMEM((NCHUNK,), _I32),
            pltpu.MemorySpace.VMEM((NCHUNK,), _I32),
            pltpu.MemorySpace.VMEM((NCHUNK, 32), _F32),
            pltpu.MemorySpace.VMEM((NCHUNK, 64), _F32),
            pltpu.MemorySpace.VMEM((NCHUNK * 4 + 16,), _F32),
            pltpu.MemorySpace.VMEM((NCHUNK, C), _F32),
            pltpu.SemaphoreType.DMA,
        ],
    )(send_p, recv_p, *hgs, *rwgs, sh_p)


def _edge_body(vec_ref, w1a_ref, w2a_ref, w3a_ref, w1b_ref, w2b_ref, w3b_ref,
               rwa0, rwa1, rwa2, rwa3, rwb0, rwb1, rwb2, rwb3, sh_ref):
    vec = vec_ref[...]
    d2 = jnp.sum(vec * vec, axis=1, keepdims=True) + 1e-12
    r = jnp.sqrt(d2)
    inv_r = 1.0 / r
    unit = vec * inv_r
    sh_ref[...] = jnp.concatenate(
        [jnp.ones((vec.shape[0], 1), _F32), np.sqrt(3.0).astype(np.float32) * unit], axis=1)
    n = (lax.broadcasted_iota(_I32, (vec.shape[0], NB), 1).astype(_F32)
         + 1.0) * (np.pi / RMAX)
    arg = r * n
    u = r * (1.0 / RMAX)
    u5 = u * u * u * u * u
    env = 1.0 - 21.0 * u5 + 35.0 * u5 * u - 15.0 * u5 * u * u
    env = jnp.where(u < 1.0, env, 0.0)
    pref = np.sqrt(2.0 / RMAX).astype(np.float32)
    ef = (pref * jnp.sin(arg)) * (inv_r * env)
    for w1, w2, w3, outs in ((w1a_ref, w2a_ref, w3a_ref, (rwa0, rwa1, rwa2, rwa3)),
                             (w1b_ref, w2b_ref, w3b_ref, (rwb0, rwb1, rwb2, rwb3))):
        t = C2M * _silu(jnp.dot(ef, w1[...], preferred_element_type=_F32))
        t = C2M * _silu(jnp.dot(t, w2[...], preferred_element_type=_F32))
        full = jnp.dot(t, w3[...], preferred_element_type=_F32)
        for g in range(4):
            outs[g][...] = full[:, 64 * g:64 * (g + 1)]


def _edge_pass(vec, w3p_0, w3p_1, rW1_0, rW2_0, rW1_1, rW2_1):
    return pl.pallas_call(
        _edge_body,
        grid=(EP // BE,),
        in_specs=[
            pl.BlockSpec((BE, 3), lambda i: (i, 0)),
            pl.BlockSpec((NB, 64), lambda i: (0, 0)),
            pl.BlockSpec((64, 64), lambda i: (0, 0)),
            pl.BlockSpec((64, 2 * C), lambda i: (0, 0)),
            pl.BlockSpec((NB, 64), lambda i: (0, 0)),
            pl.BlockSpec((64, 64), lambda i: (0, 0)),
            pl.BlockSpec((64, 2 * C), lambda i: (0, 0)),
        ],
        out_specs=[pl.BlockSpec((BE, 64), lambda i: (i, 0))] * 8
        + [pl.BlockSpec((BE, 4), lambda i: (i, 0))],
        out_shape=[jax.ShapeDtypeStruct((EP, 64), _F32)] * 8
        + [jax.ShapeDtypeStruct((EP, 4), _F32)],
    )(vec, rW1_0, rW2_0, w3p_0, rW1_1, rW2_1, w3p_1)


def _embed_body(na_ref, wemb_ref, e0w_ref, hg0, hg1, hg2, hg3, e_ref):
    na = na_ref[...]
    h = jnp.dot(na, wemb_ref[...], preferred_element_type=_F32)
    for g in range(4):
        (hg0, hg1, hg2, hg3)[g][...] = h[:, 32 * g:32 * (g + 1)]
    e_ref[...] = jnp.dot(na, e0w_ref[...], preferred_element_type=_F32)


def _embed(node_attrs, W_emb, E0_w):
    return pl.pallas_call(
        _embed_body,
        grid=(N // BN,),
        in_specs=[
            pl.BlockSpec((BN, NE), lambda i: (i, 0)),
            pl.BlockSpec((NE, C), lambda i: (0, 0)),
            pl.BlockSpec((NE, 1), lambda i: (0, 0)),
        ],
        out_specs=[pl.BlockSpec((BN, 32), lambda i: (i, 0))] * 4
        + [pl.BlockSpec((BN, 1), lambda i: (i, 0))],
        out_shape=[jax.ShapeDtypeStruct((N, 32), _F32)] * 4
        + [jax.ShapeDtypeStruct((N, 1), _F32)],
    )(node_attrs, W_emb, E0_w.reshape(NE, 1))


def _inv_from_agg(agg_refs):
    pieces = []
    for g in range(4):
        a = agg_refs[g][...]
        a0 = a[:, 0:32] * (1.0 / AVG)
        sq = (a[:, 32:64] ** 2 + a[:, 64:96] ** 2 + a[:, 96:128] ** 2) * (1.0 / (AVG * AVG))
        pieces.append(a0 + sq)
    return jnp.concatenate(pieces, axis=1)


def _node0_body(a0, a1, a2, a3, wm_ref, wro_ref, hg0, hg1, hg2, hg3, e_ref):
    inv = _inv_from_agg((a0, a1, a2, a3))
    h = jnp.dot(inv, wm_ref[...], preferred_element_type=_F32)
    for g in range(4):
        (hg0, hg1, hg2, hg3)[g][...] = h[:, 32 * g:32 * (g + 1)]
    e_ref[...] = jnp.dot(h, wro_ref[...], preferred_element_type=_F32)


def _node0(aggs, Wmix, Wro):
    return pl.pallas_call(
        _node0_body,
        grid=(N // BN,),
        in_specs=[pl.BlockSpec((BN, C), lambda i: (i, 0))] * 4
        + [pl.BlockSpec((C, C), lambda i: (0, 0)),
           pl.BlockSpec((C, 1), lambda i: (0, 0))],
        out_specs=[pl.BlockSpec((BN, 32), lambda i: (i, 0))] * 4
        + [pl.BlockSpec((BN, 1), lambda i: (i, 0))],
        out_shape=[jax.ShapeDtypeStruct((N, 32), _F32)] * 4
        + [jax.ShapeDtypeStruct((N, 1), _F32)],
    )(*aggs, Wmix, Wro)


def _node1_body(a0, a1, a2, a3, wm_ref, wh_ref, wo_ref, e_ref):
    inv = _inv_from_agg((a0, a1, a2, a3))
    h = jnp.dot(inv, wm_ref[...], preferred_element_type=_F32)
    hh = C2M * _silu(jnp.dot(h, wh_ref[...], preferred_element_type=_F32))
    e_ref[...] = jnp.dot(hh, wo_ref[...], preferred_element_type=_F32)


def _node1(aggs, Wmix, Wh, Wo):
    return pl.pallas_call(
        _node1_body,
        grid=(N // BN,),
        in_specs=[pl.BlockSpec((BN, C), lambda i: (i, 0))] * 4
        + [pl.BlockSpec((C, C), lambda i: (0, 0)),
           pl.BlockSpec((C, 16), lambda i: (0, 0)),
           pl.BlockSpec((16, 1), lambda i: (0, 0))],
        out_specs=pl.BlockSpec((BN, 1), lambda i: (i, 0)),
        out_shape=jax.ShapeDtypeStruct((N, 1), _F32),
    )(*aggs, Wmix, Wh, Wo)


def kernel(node_attrs, positions, shifts, W_emb, E0_w,
           rW1_0, rW2_0, rW3_0, Wmix_0, Wro_0,
           rW1_1, rW2_1, rW3_1, Wmix_1, Wh, Wo, edge_index):
    sender = edge_index[0].astype(_I32)
    receiver = edge_index[1].astype(_I32)
    pad = EP - E
    send_p = jnp.concatenate([sender, jnp.zeros((pad,), _I32)])
    recv_g = jnp.concatenate([receiver, jnp.zeros((pad,), _I32)])
    recv_m = jnp.concatenate([receiver, jnp.full((pad,), DUMMY, _I32)])
    shifts_p = jnp.concatenate([shifts, jnp.zeros((pad, 3), _F32)], axis=0).reshape(EP * 3)

    perm = np.array([(32 * g + cp) * 2 + path
                     for g in range(4) for path in range(2) for cp in range(32)])
    w3p_0 = rW3_0[:, perm]
    w3p_1 = rW3_1[:, perm]

    vec = _geometry(positions.reshape(N * 3), send_p, recv_g, shifts_p).reshape(EP, 3)

    eouts = _edge_pass(vec, w3p_0, w3p_1, rW1_0, rW2_0, rW1_1, rW2_1)
    rwg0, rwg1, sh_p = eouts[0:4], eouts[4:8], eouts[8].reshape(EP * 4)

    *hgs, e0 = _embed(node_attrs, W_emb, E0_w)
    e = e0[:, 0]

    aggs = _message(send_p, recv_m, hgs, rwg0, sh_p)
    *hgs, ep1 = _node0(aggs, Wmix_0, Wro_0)
    e = e + ep1[:, 0]

    aggs = _message(send_p, recv_m, hgs, rwg1, sh_p)
    e = e + _node1(aggs, Wmix_1, Wh, Wo)[:, 0]
    return e


# speedup vs baseline: 1.3555x; 1.3555x over previous
import functools

import jax
import jax.numpy as jnp
import numpy as np
from jax import lax
from jax.experimental import pallas as pl
from jax.experimental.pallas import tpu as pltpu
from jax.experimental.pallas import tpu_sc as plsc

N = 10000
E = 160000
NE = 4
C = 128
RMAX = 5.0
NB = 8
P = 5
AVG = 16.0
C2M = 1.6792

NCORES = 2
NSUB = 16
EP = 163840
GCH = EP // (NCORES * NSUB)
MCH = EP // NSUB
NCHUNK = 128
NACC = 10240
DUMMY = N
NWB = NACC // NSUB

BE = 2048
BN = 2000

_I32 = jnp.int32
_F32 = jnp.float32


def _silu(x):
    return x * jax.nn.sigmoid(x)


def _full16(v):
    return jnp.full((16,), v, _I32)


def _geom_body(pos_hbm, send_hbm, recv_hbm, shift_hbm, vec_hbm,
               posb, sbuf, rbuf, shb, vb):
    c = lax.axis_index("c")
    s = lax.axis_index("s")
    wid = s * NCORES + c
    base = wid * GCH
    pltpu.sync_copy(pos_hbm, posb)
    pltpu.sync_copy(send_hbm.at[pl.ds(base, GCH)], sbuf)
    pltpu.sync_copy(recv_hbm.at[pl.ds(base, GCH)], rbuf)
    pltpu.sync_copy(shift_hbm.at[pl.ds(base * 3, GCH * 3)], shb)
    iota16 = lax.iota(_I32, 16)
    three = _full16(3)

    @pl.loop(0, GCH // 16)
    def _micro(m):
        off = pl.multiple_of(m * 16, 16)
        sidx = sbuf[pl.ds(off, 16)] * three
        ridx = rbuf[pl.ds(off, 16)] * three
        lidx = (jnp.full((16,), off, _I32) + iota16) * three
        for k in range(3):
            kc = _full16(k)
            p_s = plsc.load_gather(posb, [sidx + kc])
            p_r = plsc.load_gather(posb, [ridx + kc])
            sh = plsc.load_gather(shb, [lidx + kc])
            plsc.store_scatter(vb, [lidx + kc], p_r - p_s + sh)

    pltpu.sync_copy(vb, vec_hbm.at[pl.ds(base * 3, GCH * 3)])


def _geometry(positions, send_p, recv_p, shifts_p):
    mesh = plsc.VectorSubcoreMesh(core_axis_name="c", subcore_axis_name="s")
    return pl.kernel(
        _geom_body,
        out_type=jax.ShapeDtypeStruct((EP * 3,), _F32),
        mesh=mesh,
        compiler_params=pltpu.CompilerParams(needs_layout_passes=False),
        scratch_types=[
            pltpu.MemorySpace.VMEM((N * 3,), _F32),
            pltpu.MemorySpace.VMEM((GCH,), _I32),
            pltpu.MemorySpace.VMEM((GCH,), _I32),
            pltpu.MemorySpace.VMEM((GCH * 3,), _F32),
            pltpu.MemorySpace.VMEM((GCH * 3,), _F32),
        ],
    )(positions, send_p, recv_p, shifts_p)


NCH = MCH // NCHUNK


def _msg_body(send_hbm, recv_hbm, hg0, hg1, hg2, hg3, rw0, rw1, rw2, rw3,
              sh_hbm, agg0, agg1, agg2, agg3,
              acc, sidx0, sidx1, ridx0, ridx1, shb0, shb1,
              rwb0, rwb1, hb0, hb1, msgbuf,
              semA0, semA1, semG0, semG1):
    c = lax.axis_index("c")
    s = lax.axis_index("s")
    hgs = (hg0, hg1, hg2, hg3)
    rws = (rw0, rw1, rw2, rw3)
    aggs = (agg0, agg1, agg2, agg3)
    sidx = (sidx0, sidx1)
    ridx = (ridx0, ridx1)
    shb = (shb0, shb1)
    rwb = (rwb0, rwb1)
    hb = (hb0, hb1)
    semA = (semA0, semA1)
    semG = (semG0, semG1)
    zero16 = jnp.zeros((16,), _F32)
    c1 = _full16(1)
    c2 = _full16(2)
    c3 = _full16(3)

    def _a_copies(j, sl, rwg):
        e0 = s * MCH + j * NCHUNK
        return (
            (send_hbm.at[pl.ds(e0, NCHUNK)], sidx[sl]),
            (recv_hbm.at[pl.ds(e0, NCHUNK)], ridx[sl]),
            (sh_hbm.at[pl.ds(e0 * 4, NCHUNK * 4)], shb[sl].at[pl.ds(0, NCHUNK * 4)]),
            (rwg.at[pl.ds(e0, NCHUNK)], rwb[sl]),
        )

    def _issue_a(j, sl, rwg):
        for src, dst in _a_copies(j, sl, rwg):
            pltpu.async_copy(src, dst, semA[sl])

    def _wait_a(j, sl, rwg):
        for src, dst in _a_copies(j, sl, rwg):
            pltpu.make_async_copy(src, dst, semA[sl]).wait()

    def _issue_g(sl, hg):
        pltpu.async_copy(hg.at[sidx[sl]], hb[sl], semG[sl])

    def _wait_g(sl, hg):
        pltpu.make_async_copy(hg.at[sidx[sl]], hb[sl], semG[sl]).wait()

    def _compute(sl):
        hbuf, rwbuf, shbuf = hb[sl], rwb[sl], shb[sl]

        @pl.loop(0, NCHUNK)
        def _edge(i):
            ha = hbuf[i, 0:16]
            hb_ = hbuf[i, 16:32]
            r0a = rwbuf[i, 0:16]
            r0b = rwbuf[i, 16:32]
            r1a = rwbuf[i, 32:48]
            r1b = rwbuf[i, 48:64]
            sv = shbuf[pl.ds(i * 4, 16)]
            s1 = sv[c1]
            s2 = sv[c2]
            s3 = sv[c3]
            h1a = ha * r1a
            h1b = hb_ * r1b
            msgbuf[i, 0:16] = ha * r0a
            msgbuf[i, 16:32] = hb_ * r0b
            msgbuf[i, 32:48] = h1a * s1
            msgbuf[i, 48:64] = h1b * s1
            msgbuf[i, 64:80] = h1a * s2
            msgbuf[i, 80:96] = h1b * s2
            msgbuf[i, 96:112] = h1a * s3
            msgbuf[i, 112:128] = h1b * s3

    def _pipeline(hg, rwg):
        _issue_a(0, 0, rwg)
        _wait_a(0, 0, rwg)
        _issue_g(0, hg)
        _issue_a(1, 1, rwg)

        @pl.loop(0, NCH // 2)
        def _chunk(jj):
            for half in range(2):
                sl, o = half, 1 - half
                j = jj * 2 + half

                @pl.when(j + 1 < NCH)
                def _():
                    _wait_a(j + 1, o, rwg)
                    _issue_g(o, hg)

                _wait_g(sl, hg)
                _compute(sl)
                pltpu.sync_copy(msgbuf, acc.at[ridx[sl]], add=True)

                @pl.when(j + 2 < NCH)
                def _():
                    _issue_a(j + 2, sl, rwg)

    for p in range(2):
        @pl.loop(0, NCHUNK)
        def _z(i):
            for jz in range(8):
                msgbuf[i, 16 * jz:16 * (jz + 1)] = zero16

        for z in range(NACC // (NSUB * NCHUNK)):
            pltpu.sync_copy(msgbuf, acc.at[pl.ds((s * (NACC // (NSUB * NCHUNK)) + z) * NCHUNK, NCHUNK)])
        plsc.subcore_barrier()

        for cs in range(NCORES):
            g = NCORES * cs + p

            @pl.when(c == cs)
            def _():
                _pipeline(hgs[g], rws[g])

        plsc.subcore_barrier()
        for cs in range(NCORES):
            g = NCORES * cs + p

            @pl.when(c == cs)
            def _():
                pltpu.sync_copy(acc.at[pl.ds(s * NWB, NWB)],
                                aggs[g].at[pl.ds(s * NWB, NWB)])
        plsc.subcore_barrier()


def _message(send_p, recv_p, hgs, rwgs, sh_p):
    mesh = plsc.VectorSubcoreMesh(core_axis_name="c", subcore_axis_name="s")
    return pl.kernel(
        _msg_body,
        out_type=[jax.ShapeDtypeStruct((NACC, C), _F32)] * 4,
        mesh=mesh,
        compiler_params=pltpu.CompilerParams(needs_layout_passes=False,
                                             use_tc_tiling_on_sc=False),
        scratch_types=[
            pltpu.MemorySpace.VMEM_SHARED((NACC, C), _F32),
            pltpu.MemorySpace.VMEM((NCHUNK,), _I32),
            pltpu.MemorySpace.VMEM((NCHUNK,), _I32),
            pltpu.MemorySpace.VMEM((NCHUNK,), _I32),
            pltpu.MemorySpace.VMEM((NCHUNK,), _I32),
            pltpu.MemorySpace.VMEM((NCHUNK * 4 + 16,), _F32),
            pltpu.MemorySpace.VMEM((NCHUNK * 4 + 16,), _F32),
            pltpu.MemorySpace.VMEM((NCHUNK, 64), _F32),
            pltpu.MemorySpace.VMEM((NCHUNK, 64), _F32),
            pltpu.MemorySpace.VMEM((NCHUNK, 32), _F32),
            pltpu.MemorySpace.VMEM((NCHUNK, 32), _F32),
            pltpu.MemorySpace.VMEM((NCHUNK, C), _F32),
            pltpu.SemaphoreType.DMA,
            pltpu.SemaphoreType.DMA,
            pltpu.SemaphoreType.DMA,
            pltpu.SemaphoreType.DMA,
        ],
    )(send_p, recv_p, *hgs, *rwgs, sh_p)


def _edge_body(vec_ref, w1a_ref, w2a_ref, w3a_ref, w1b_ref, w2b_ref, w3b_ref,
               rwa0, rwa1, rwa2, rwa3, rwb0, rwb1, rwb2, rwb3, sh_ref):
    vec = vec_ref[...]
    d2 = jnp.sum(vec * vec, axis=1, keepdims=True) + 1e-12
    r = jnp.sqrt(d2)
    inv_r = 1.0 / r
    unit = vec * inv_r
    sh_ref[...] = jnp.concatenate(
        [jnp.ones((vec.shape[0], 1), _F32), np.sqrt(3.0).astype(np.float32) * unit], axis=1)
    n = (lax.broadcasted_iota(_I32, (vec.shape[0], NB), 1).astype(_F32)
         + 1.0) * (np.pi / RMAX)
    arg = r * n
    u = r * (1.0 / RMAX)
    u5 = u * u * u * u * u
    env = 1.0 - 21.0 * u5 + 35.0 * u5 * u - 15.0 * u5 * u * u
    env = jnp.where(u < 1.0, env, 0.0)
    pref = np.sqrt(2.0 / RMAX).astype(np.float32)
    ef = (pref * jnp.sin(arg)) * (inv_r * env)
    for w1, w2, w3, outs in ((w1a_ref, w2a_ref, w3a_ref, (rwa0, rwa1, rwa2, rwa3)),
                             (w1b_ref, w2b_ref, w3b_ref, (rwb0, rwb1, rwb2, rwb3))):
        t = C2M * _silu(jnp.dot(ef, w1[...], preferred_element_type=_F32))
        t = C2M * _silu(jnp.dot(t, w2[...], preferred_element_type=_F32))
        full = jnp.dot(t, w3[...], preferred_element_type=_F32)
        for g in range(4):
            outs[g][...] = full[:, 64 * g:64 * (g + 1)]


def _edge_pass(vec, w3p_0, w3p_1, rW1_0, rW2_0, rW1_1, rW2_1):
    return pl.pallas_call(
        _edge_body,
        grid=(EP // BE,),
        in_specs=[
            pl.BlockSpec((BE, 3), lambda i: (i, 0)),
            pl.BlockSpec((NB, 64), lambda i: (0, 0)),
            pl.BlockSpec((64, 64), lambda i: (0, 0)),
            pl.BlockSpec((64, 2 * C), lambda i: (0, 0)),
            pl.BlockSpec((NB, 64), lambda i: (0, 0)),
            pl.BlockSpec((64, 64), lambda i: (0, 0)),
            pl.BlockSpec((64, 2 * C), lambda i: (0, 0)),
        ],
        out_specs=[pl.BlockSpec((BE, 64), lambda i: (i, 0))] * 8
        + [pl.BlockSpec((BE, 4), lambda i: (i, 0))],
        out_shape=[jax.ShapeDtypeStruct((EP, 64), _F32)] * 8
        + [jax.ShapeDtypeStruct((EP, 4), _F32)],
    )(vec, rW1_0, rW2_0, w3p_0, rW1_1, rW2_1, w3p_1)


def _embed_body(na_ref, wemb_ref, e0w_ref, hg0, hg1, hg2, hg3, e_ref):
    na = na_ref[...]
    h = jnp.dot(na, wemb_ref[...], preferred_element_type=_F32)
    for g in range(4):
        (hg0, hg1, hg2, hg3)[g][...] = h[:, 32 * g:32 * (g + 1)]
    e_ref[...] = jnp.dot(na, e0w_ref[...], preferred_element_type=_F32)


def _embed(node_attrs, W_emb, E0_w):
    return pl.pallas_call(
        _embed_body,
        grid=(N // BN,),
        in_specs=[
            pl.BlockSpec((BN, NE), lambda i: (i, 0)),
            pl.BlockSpec((NE, C), lambda i: (0, 0)),
            pl.BlockSpec((NE, 1), lambda i: (0, 0)),
        ],
        out_specs=[pl.BlockSpec((BN, 32), lambda i: (i, 0))] * 4
        + [pl.BlockSpec((BN, 1), lambda i: (i, 0))],
        out_shape=[jax.ShapeDtypeStruct((N, 32), _F32)] * 4
        + [jax.ShapeDtypeStruct((N, 1), _F32)],
    )(node_attrs, W_emb, E0_w.reshape(NE, 1))


def _inv_from_agg(agg_refs):
    pieces = []
    for g in range(4):
        a = agg_refs[g][...]
        a0 = a[:, 0:32] * (1.0 / AVG)
        sq = (a[:, 32:64] ** 2 + a[:, 64:96] ** 2 + a[:, 96:128] ** 2) * (1.0 / (AVG * AVG))
        pieces.append(a0 + sq)
    return jnp.concatenate(pieces, axis=1)


def _node0_body(a0, a1, a2, a3, wm_ref, wro_ref, hg0, hg1, hg2, hg3, e_ref):
    inv = _inv_from_agg((a0, a1, a2, a3))
    h = jnp.dot(inv, wm_ref[...], preferred_element_type=_F32)
    for g in range(4):
        (hg0, hg1, hg2, hg3)[g][...] = h[:, 32 * g:32 * (g + 1)]
    e_ref[...] = jnp.dot(h, wro_ref[...], preferred_element_type=_F32)


def _node0(aggs, Wmix, Wro):
    return pl.pallas_call(
        _node0_body,
        grid=(N // BN,),
        in_specs=[pl.BlockSpec((BN, C), lambda i: (i, 0))] * 4
        + [pl.BlockSpec((C, C), lambda i: (0, 0)),
           pl.BlockSpec((C, 1), lambda i: (0, 0))],
        out_specs=[pl.BlockSpec((BN, 32), lambda i: (i, 0))] * 4
        + [pl.BlockSpec((BN, 1), lambda i: (i, 0))],
        out_shape=[jax.ShapeDtypeStruct((N, 32), _F32)] * 4
        + [jax.ShapeDtypeStruct((N, 1), _F32)],
    )(*aggs, Wmix, Wro)


def _node1_body(a0, a1, a2, a3, wm_ref, wh_ref, wo_ref, e_ref):
    inv = _inv_from_agg((a0, a1, a2, a3))
    h = jnp.dot(inv, wm_ref[...], preferred_element_type=_F32)
    hh = C2M * _silu(jnp.dot(h, wh_ref[...], preferred_element_type=_F32))
    e_ref[...] = jnp.dot(hh, wo_ref[...], preferred_element_type=_F32)


def _node1(aggs, Wmix, Wh, Wo):
    return pl.pallas_call(
        _node1_body,
        grid=(N // BN,),
        in_specs=[pl.BlockSpec((BN, C), lambda i: (i, 0))] * 4
        + [pl.BlockSpec((C, C), lambda i: (0, 0)),
           pl.BlockSpec((C, 16), lambda i: (0, 0)),
           pl.BlockSpec((16, 1), lambda i: (0, 0))],
        out_specs=pl.BlockSpec((BN, 1), lambda i: (i, 0)),
        out_shape=jax.ShapeDtypeStruct((N, 1), _F32),
    )(*aggs, Wmix, Wh, Wo)


def kernel(node_attrs, positions, shifts, W_emb, E0_w,
           rW1_0, rW2_0, rW3_0, Wmix_0, Wro_0,
           rW1_1, rW2_1, rW3_1, Wmix_1, Wh, Wo, edge_index):
    sender = edge_index[0].astype(_I32)
    receiver = edge_index[1].astype(_I32)
    pad = EP - E
    send_p = jnp.concatenate([sender, jnp.zeros((pad,), _I32)])
    recv_g = jnp.concatenate([receiver, jnp.zeros((pad,), _I32)])
    recv_m = jnp.concatenate([receiver, jnp.full((pad,), DUMMY, _I32)])
    shifts_p = jnp.concatenate([shifts, jnp.zeros((pad, 3), _F32)], axis=0).reshape(EP * 3)

    perm = np.array([(32 * g + cp) * 2 + path
                     for g in range(4) for path in range(2) for cp in range(32)])
    w3p_0 = rW3_0[:, perm]
    w3p_1 = rW3_1[:, perm]

    vec = _geometry(positions.reshape(N * 3), send_p, recv_g, shifts_p).reshape(EP, 3)

    eouts = _edge_pass(vec, w3p_0, w3p_1, rW1_0, rW2_0, rW1_1, rW2_1)
    rwg0, rwg1, sh_p = eouts[0:4], eouts[4:8], eouts[8].reshape(EP * 4)

    *hgs, e0 = _embed(node_attrs, W_emb, E0_w)
    e = e0[:, 0]

    aggs = _message(send_p, recv_m, hgs, rwg0, sh_p)
    *hgs, ep1 = _node0(aggs, Wmix_0, Wro_0)
    e = e + ep1[:, 0]

    aggs = _message(send_p, recv_m, hgs, rwg1, sh_p)
    e = e + _node1(aggs, Wmix_1, Wh, Wo)[:, 0]
    return e


# speedup vs baseline: 1.6874x; 1.2449x over previous
import functools

import jax
import jax.numpy as jnp
import numpy as np
from jax import lax
from jax.experimental import pallas as pl
from jax.experimental.pallas import tpu as pltpu
from jax.experimental.pallas import tpu_sc as plsc

N = 10000
E = 160000
NE = 4
C = 128
RMAX = 5.0
NB = 8
P = 5
AVG = 16.0
C2M = 1.6792

NCORES = 2
NSUB = 16
EP = 163840
GCH = EP // (NCORES * NSUB)
MCH = EP // NSUB
NCHUNK = 128
NACC = 10240
DUMMY = N
NWB = NACC // NSUB

BE = 2048
BN = 2000

_I32 = jnp.int32
_F32 = jnp.float32


def _silu(x):
    return x * jax.nn.sigmoid(x)


def _full16(v):
    return jnp.full((16,), v, _I32)


def _geom_body(pos_hbm, send_hbm, recv_hbm, vec_hbm,
               posb, sbuf, rbuf, vb):
    c = lax.axis_index("c")
    s = lax.axis_index("s")
    wid = s * NCORES + c
    base = wid * GCH
    pltpu.sync_copy(pos_hbm, posb)
    pltpu.sync_copy(send_hbm.at[pl.ds(base, GCH)], sbuf)
    pltpu.sync_copy(recv_hbm.at[pl.ds(base, GCH)], rbuf)
    iota16 = lax.iota(_I32, 16)
    three = _full16(3)

    @pl.loop(0, GCH // 16)
    def _micro(m):
        off = pl.multiple_of(m * 16, 16)
        sidx = sbuf[pl.ds(off, 16)] * three
        ridx = rbuf[pl.ds(off, 16)] * three
        lidx = (jnp.full((16,), off, _I32) + iota16) * three
        for k in range(3):
            kc = _full16(k)
            p_s = plsc.load_gather(posb, [sidx + kc])
            p_r = plsc.load_gather(posb, [ridx + kc])
            plsc.store_scatter(vb, [lidx + kc], p_r - p_s)

    pltpu.sync_copy(vb, vec_hbm.at[pl.ds(base * 3, GCH * 3)])


def _geometry(positions, send_p, recv_p):
    mesh = plsc.VectorSubcoreMesh(core_axis_name="c", subcore_axis_name="s")
    return pl.kernel(
        _geom_body,
        out_type=jax.ShapeDtypeStruct((EP * 3,), _F32),
        mesh=mesh,
        compiler_params=pltpu.CompilerParams(needs_layout_passes=False),
        scratch_types=[
            pltpu.MemorySpace.VMEM((N * 3,), _F32),
            pltpu.MemorySpace.VMEM((GCH,), _I32),
            pltpu.MemorySpace.VMEM((GCH,), _I32),
            pltpu.MemorySpace.VMEM((GCH * 3,), _F32),
        ],
    )(positions, send_p, recv_p)


NCH = MCH // NCHUNK


def _msg_body(send_hbm, recv_hbm, hg0, hg1, hg2, hg3, rwA, rwB,
              sh_hbm, agg0, agg1, agg2, agg3,
              acc, sidx0, sidx1, ridx0, ridx1, shb0, shb1,
              rwb0, rwb1, hb0, hb1, msgbuf,
              semA0, semA1, semG0, semG1):
    c = lax.axis_index("c")
    s = lax.axis_index("s")
    hgs = (hg0, hg1, hg2, hg3)
    rwp = (rwA, rwB)
    aggs = (agg0, agg1, agg2, agg3)
    sidx = (sidx0, sidx1)
    ridx = (ridx0, ridx1)
    shb = (shb0, shb1)
    rwb = (rwb0, rwb1)
    hb = (hb0, hb1)
    semA = (semA0, semA1)
    semG = (semG0, semG1)
    zero16 = jnp.zeros((16,), _F32)
    c1 = _full16(1)
    c2 = _full16(2)
    c3 = _full16(3)

    def _a_copies(j, sl, rwg):
        src, off = rwg
        e0 = s * MCH + j * NCHUNK
        return (
            (send_hbm.at[pl.ds(e0, NCHUNK)], sidx[sl]),
            (recv_hbm.at[pl.ds(e0, NCHUNK)], ridx[sl]),
            (sh_hbm.at[pl.ds(e0 * 4, NCHUNK * 4)], shb[sl].at[pl.ds(0, NCHUNK * 4)]),
            (src.at[pl.ds(e0, NCHUNK), pl.ds(off, 64)], rwb[sl]),
        )

    def _issue_a(j, sl, rwg):
        for src, dst in _a_copies(j, sl, rwg):
            pltpu.async_copy(src, dst, semA[sl])

    def _wait_a(j, sl, rwg):
        for src, dst in _a_copies(j, sl, rwg):
            pltpu.make_async_copy(src, dst, semA[sl]).wait()

    def _issue_g(sl, hg):
        pltpu.async_copy(hg.at[sidx[sl]], hb[sl], semG[sl])

    def _wait_g(sl, hg):
        pltpu.make_async_copy(hg.at[sidx[sl]], hb[sl], semG[sl]).wait()

    def _compute(sl):
        hbuf, rwbuf, shbuf = hb[sl], rwb[sl], shb[sl]

        @pl.loop(0, NCHUNK)
        def _edge(i):
            ha = hbuf[i, 0:16]
            hb_ = hbuf[i, 16:32]
            r0a = rwbuf[i, 0:16]
            r0b = rwbuf[i, 16:32]
            r1a = rwbuf[i, 32:48]
            r1b = rwbuf[i, 48:64]
            sv = shbuf[pl.ds(i * 4, 16)]
            s1 = sv[c1]
            s2 = sv[c2]
            s3 = sv[c3]
            h1a = ha * r1a
            h1b = hb_ * r1b
            msgbuf[i, 0:16] = ha * r0a
            msgbuf[i, 16:32] = hb_ * r0b
            msgbuf[i, 32:48] = h1a * s1
            msgbuf[i, 48:64] = h1b * s1
            msgbuf[i, 64:80] = h1a * s2
            msgbuf[i, 80:96] = h1b * s2
            msgbuf[i, 96:112] = h1a * s3
            msgbuf[i, 112:128] = h1b * s3

    def _pipeline(hg, rwg):
        _issue_a(0, 0, rwg)
        _wait_a(0, 0, rwg)
        _issue_g(0, hg)
        _issue_a(1, 1, rwg)

        @pl.loop(0, NCH // 2)
        def _chunk(jj):
            for half in range(2):
                sl, o = half, 1 - half
                j = jj * 2 + half

                @pl.when(j + 1 < NCH)
                def _():
                    _wait_a(j + 1, o, rwg)
                    _issue_g(o, hg)

                _wait_g(sl, hg)
                _compute(sl)
                pltpu.sync_copy(msgbuf, acc.at[ridx[sl]], add=True)

                @pl.when(j + 2 < NCH)
                def _():
                    _issue_a(j + 2, sl, rwg)

    for p in range(2):
        @pl.loop(0, NCHUNK)
        def _z(i):
            for jz in range(8):
                msgbuf[i, 16 * jz:16 * (jz + 1)] = zero16

        for z in range(NACC // (NSUB * NCHUNK)):
            pltpu.sync_copy(msgbuf, acc.at[pl.ds((s * (NACC // (NSUB * NCHUNK)) + z) * NCHUNK, NCHUNK)])
        plsc.subcore_barrier()

        for cs in range(NCORES):
            g = NCORES * cs + p

            @pl.when(c == cs)
            def _():
                _pipeline(hgs[g], (rwp[p], 64 * cs))

        plsc.subcore_barrier()
        for cs in range(NCORES):
            g = NCORES * cs + p

            @pl.when(c == cs)
            def _():
                pltpu.sync_copy(acc.at[pl.ds(s * NWB, NWB)],
                                aggs[g].at[pl.ds(s * NWB, NWB)])
        plsc.subcore_barrier()


def _message(send_p, recv_p, hgs, rwpair, sh_p):
    mesh = plsc.VectorSubcoreMesh(core_axis_name="c", subcore_axis_name="s")
    return pl.kernel(
        _msg_body,
        out_type=[jax.ShapeDtypeStruct((NACC, C), _F32)] * 4,
        mesh=mesh,
        compiler_params=pltpu.CompilerParams(needs_layout_passes=False,
                                             use_tc_tiling_on_sc=False),
        scratch_types=[
            pltpu.MemorySpace.VMEM_SHARED((NACC, C), _F32),
            pltpu.MemorySpace.VMEM((NCHUNK,), _I32),
            pltpu.MemorySpace.VMEM((NCHUNK,), _I32),
            pltpu.MemorySpace.VMEM((NCHUNK,), _I32),
            pltpu.MemorySpace.VMEM((NCHUNK,), _I32),
            pltpu.MemorySpace.VMEM((NCHUNK * 4 + 16,), _F32),
            pltpu.MemorySpace.VMEM((NCHUNK * 4 + 16,), _F32),
            pltpu.MemorySpace.VMEM((NCHUNK, 64), _F32),
            pltpu.MemorySpace.VMEM((NCHUNK, 64), _F32),
            pltpu.MemorySpace.VMEM((NCHUNK, 32), _F32),
            pltpu.MemorySpace.VMEM((NCHUNK, 32), _F32),
            pltpu.MemorySpace.VMEM((NCHUNK, C), _F32),
            pltpu.SemaphoreType.DMA,
            pltpu.SemaphoreType.DMA,
            pltpu.SemaphoreType.DMA,
            pltpu.SemaphoreType.DMA,
        ],
    )(send_p, recv_p, *hgs, *rwpair, sh_p)


def _edge_body(vec_ref, w1a_ref, w2a_ref, w3a_ref, w1b_ref, w2b_ref, w3b_ref,
               rwA0, rwB0, rwA1, rwB1, sh_ref):
    vec = vec_ref[...]
    d2 = jnp.sum(vec * vec, axis=1, keepdims=True) + 1e-12
    r = jnp.sqrt(d2)
    inv_r = 1.0 / r
    unit = vec * inv_r
    sh_ref[...] = jnp.concatenate(
        [jnp.ones((vec.shape[0], 1), _F32), np.sqrt(3.0).astype(np.float32) * unit], axis=1)
    n = (lax.broadcasted_iota(_I32, (vec.shape[0], NB), 1).astype(_F32)
         + 1.0) * (np.pi / RMAX)
    arg = r * n
    u = r * (1.0 / RMAX)
    u5 = u * u * u * u * u
    env = 1.0 - 21.0 * u5 + 35.0 * u5 * u - 15.0 * u5 * u * u
    env = jnp.where(u < 1.0, env, 0.0)
    pref = np.sqrt(2.0 / RMAX).astype(np.float32)
    ef = (pref * jnp.sin(arg)) * (inv_r * env)
    for w1, w2, w3, outs in ((w1a_ref, w2a_ref, w3a_ref, (rwA0, rwB0)),
                             (w1b_ref, w2b_ref, w3b_ref, (rwA1, rwB1))):
        t = C2M * _silu(jnp.dot(ef, w1[...], preferred_element_type=_F32))
        t = C2M * _silu(jnp.dot(t, w2[...], preferred_element_type=_F32))
        full = jnp.dot(t, w3[...], preferred_element_type=_F32)
        outs[0][...] = full[:, 0:128]
        outs[1][...] = full[:, 128:256]


def _edge_pass(vec, w3p_0, w3p_1, rW1_0, rW2_0, rW1_1, rW2_1):
    return pl.pallas_call(
        _edge_body,
        grid=(EP // BE,),
        in_specs=[
            pl.BlockSpec((BE, 3), lambda i: (i, 0)),
            pl.BlockSpec((NB, 64), lambda i: (0, 0)),
            pl.BlockSpec((64, 64), lambda i: (0, 0)),
            pl.BlockSpec((64, 2 * C), lambda i: (0, 0)),
            pl.BlockSpec((NB, 64), lambda i: (0, 0)),
            pl.BlockSpec((64, 64), lambda i: (0, 0)),
            pl.BlockSpec((64, 2 * C), lambda i: (0, 0)),
        ],
        out_specs=[pl.BlockSpec((BE, 2 * 64), lambda i: (i, 0))] * 4
        + [pl.BlockSpec((BE, 4), lambda i: (i, 0))],
        out_shape=[jax.ShapeDtypeStruct((EP, 2 * 64), _F32)] * 4
        + [jax.ShapeDtypeStruct((EP, 4), _F32)],
    )(vec, rW1_0, rW2_0, w3p_0, rW1_1, rW2_1, w3p_1)


def _embed_body(na_ref, wemb_ref, e0w_ref, hg0, hg1, hg2, hg3, e_ref):
    na = na_ref[...]
    h = jnp.dot(na, wemb_ref[...], preferred_element_type=_F32)
    for g in range(4):
        (hg0, hg1, hg2, hg3)[g][...] = h[:, 32 * g:32 * (g + 1)]
    e_ref[...] = jnp.dot(na, e0w_ref[...], preferred_element_type=_F32)


def _embed(node_attrs, W_emb, E0_w):
    return pl.pallas_call(
        _embed_body,
        grid=(N // BN,),
        in_specs=[
            pl.BlockSpec((BN, NE), lambda i: (i, 0)),
            pl.BlockSpec((NE, C), lambda i: (0, 0)),
            pl.BlockSpec((NE, 1), lambda i: (0, 0)),
        ],
        out_specs=[pl.BlockSpec((BN, 32), lambda i: (i, 0))] * 4
        + [pl.BlockSpec((BN, 1), lambda i: (i, 0))],
        out_shape=[jax.ShapeDtypeStruct((N, 32), _F32)] * 4
        + [jax.ShapeDtypeStruct((N, 1), _F32)],
    )(node_attrs, W_emb, E0_w.reshape(NE, 1))


def _inv_from_agg(agg_refs):
    pieces = []
    for g in range(4):
        a = agg_refs[g][...]
        a0 = a[:, 0:32] * (1.0 / AVG)
        sq = (a[:, 32:64] ** 2 + a[:, 64:96] ** 2 + a[:, 96:128] ** 2) * (1.0 / (AVG * AVG))
        pieces.append(a0 + sq)
    return jnp.concatenate(pieces, axis=1)


def _node0_body(a0, a1, a2, a3, wm_ref, wro_ref, hg0, hg1, hg2, hg3, e_ref):
    inv = _inv_from_agg((a0, a1, a2, a3))
    h = jnp.dot(inv, wm_ref[...], preferred_element_type=_F32)
    for g in range(4):
        (hg0, hg1, hg2, hg3)[g][...] = h[:, 32 * g:32 * (g + 1)]
    e_ref[...] = jnp.dot(h, wro_ref[...], preferred_element_type=_F32)


def _node0(aggs, Wmix, Wro):
    return pl.pallas_call(
        _node0_body,
        grid=(N // BN,),
        in_specs=[pl.BlockSpec((BN, C), lambda i: (i, 0))] * 4
        + [pl.BlockSpec((C, C), lambda i: (0, 0)),
           pl.BlockSpec((C, 1), lambda i: (0, 0))],
        out_specs=[pl.BlockSpec((BN, 32), lambda i: (i, 0))] * 4
        + [pl.BlockSpec((BN, 1), lambda i: (i, 0))],
        out_shape=[jax.ShapeDtypeStruct((N, 32), _F32)] * 4
        + [jax.ShapeDtypeStruct((N, 1), _F32)],
    )(*aggs, Wmix, Wro)


def _node1_body(a0, a1, a2, a3, wm_ref, wh_ref, wo_ref, e_ref):
    inv = _inv_from_agg((a0, a1, a2, a3))
    h = jnp.dot(inv, wm_ref[...], preferred_element_type=_F32)
    hh = C2M * _silu(jnp.dot(h, wh_ref[...], preferred_element_type=_F32))
    e_ref[...] = jnp.dot(hh, wo_ref[...], preferred_element_type=_F32)


def _node1(aggs, Wmix, Wh, Wo):
    return pl.pallas_call(
        _node1_body,
        grid=(N // BN,),
        in_specs=[pl.BlockSpec((BN, C), lambda i: (i, 0))] * 4
        + [pl.BlockSpec((C, C), lambda i: (0, 0)),
           pl.BlockSpec((C, 16), lambda i: (0, 0)),
           pl.BlockSpec((16, 1), lambda i: (0, 0))],
        out_specs=pl.BlockSpec((BN, 1), lambda i: (i, 0)),
        out_shape=jax.ShapeDtypeStruct((N, 1), _F32),
    )(*aggs, Wmix, Wh, Wo)


def kernel(node_attrs, positions, shifts, W_emb, E0_w,
           rW1_0, rW2_0, rW3_0, Wmix_0, Wro_0,
           rW1_1, rW2_1, rW3_1, Wmix_1, Wh, Wo, edge_index):
    sender = edge_index[0].astype(_I32)
    receiver = edge_index[1].astype(_I32)
    pad = EP - E
    send_p = jnp.concatenate([sender, jnp.zeros((pad,), _I32)])
    recv_g = jnp.concatenate([receiver, jnp.zeros((pad,), _I32)])
    recv_m = jnp.concatenate([receiver, jnp.full((pad,), DUMMY, _I32)])

    perm = np.array([(32 * g + cp) * 2 + path
                     for g in (0, 2, 1, 3) for path in range(2) for cp in range(32)])
    w3p_0 = rW3_0[:, perm]
    w3p_1 = rW3_1[:, perm]

    vec = _geometry(positions.reshape(N * 3), send_p, recv_g).reshape(EP, 3)

    eouts = _edge_pass(vec, w3p_0, w3p_1, rW1_0, rW2_0, rW1_1, rW2_1)
    rwg0, rwg1, sh_p = eouts[0:2], eouts[2:4], eouts[4].reshape(EP * 4)

    *hgs, e0 = _embed(node_attrs, W_emb, E0_w)
    e = e0[:, 0]

    aggs = _message(send_p, recv_m, hgs, rwg0, sh_p)
    *hgs, ep1 = _node0(aggs, Wmix_0, Wro_0)
    e = e + ep1[:, 0]

    aggs = _message(send_p, recv_m, hgs, rwg1, sh_p)
    e = e + _node1(aggs, Wmix_1, Wh, Wo)[:, 0]
    return e


# speedup vs baseline: 1.7021x; 1.0087x over previous
import functools

import jax
import jax.numpy as jnp
import numpy as np
from jax import lax
from jax.experimental import pallas as pl
from jax.experimental.pallas import tpu as pltpu
from jax.experimental.pallas import tpu_sc as plsc

N = 10000
E = 160000
NE = 4
C = 128
RMAX = 5.0
NB = 8
P = 5
AVG = 16.0
C2M = 1.6792

NCORES = 2
NSUB = 16
EP = 163840
GCH = EP // (NCORES * NSUB)
MCH = EP // NSUB
NCHUNK = 64
NACC = 10240
DUMMY = N
NWB = NACC // NSUB

BE = 2048
BN = 2000

_I32 = jnp.int32
_F32 = jnp.float32


def _silu(x):
    return x * jax.nn.sigmoid(x)


def _full16(v):
    return jnp.full((16,), v, _I32)


def _geom_body(pos_hbm, send_hbm, recv_hbm, vec_hbm,
               posb, sbuf, rbuf, vb):
    c = lax.axis_index("c")
    s = lax.axis_index("s")
    wid = s * NCORES + c
    base = wid * GCH
    pltpu.sync_copy(pos_hbm, posb)
    pltpu.sync_copy(send_hbm.at[pl.ds(base, GCH)], sbuf)
    pltpu.sync_copy(recv_hbm.at[pl.ds(base, GCH)], rbuf)
    iota16 = lax.iota(_I32, 16)
    three = _full16(3)

    @pl.loop(0, GCH // 16)
    def _micro(m):
        off = pl.multiple_of(m * 16, 16)
        sidx = sbuf[pl.ds(off, 16)] * three
        ridx = rbuf[pl.ds(off, 16)] * three
        lidx = (jnp.full((16,), off, _I32) + iota16) * three
        for k in range(3):
            kc = _full16(k)
            p_s = plsc.load_gather(posb, [sidx + kc])
            p_r = plsc.load_gather(posb, [ridx + kc])
            plsc.store_scatter(vb, [lidx + kc], p_r - p_s)

    pltpu.sync_copy(vb, vec_hbm.at[pl.ds(base * 3, GCH * 3)])


def _geometry(positions, send_p, recv_p):
    mesh = plsc.VectorSubcoreMesh(core_axis_name="c", subcore_axis_name="s")
    return pl.kernel(
        _geom_body,
        out_type=jax.ShapeDtypeStruct((EP * 3,), _F32),
        mesh=mesh,
        compiler_params=pltpu.CompilerParams(needs_layout_passes=False),
        scratch_types=[
            pltpu.MemorySpace.VMEM((N * 3,), _F32),
            pltpu.MemorySpace.VMEM((GCH,), _I32),
            pltpu.MemorySpace.VMEM((GCH,), _I32),
            pltpu.MemorySpace.VMEM((GCH * 3,), _F32),
        ],
    )(positions, send_p, recv_p)


NCH = MCH // NCHUNK


def _msg_body(send_hbm, recv_hbm, hg0, hg1, hg2, hg3, rwA, rwB,
              sh_hbm, agg0, agg1, agg2, agg3,
              acc, sidx0, sidx1, ridx0, ridx1, rs0, rs1, shb0, shb1,
              rwb0, rwb1, hb0, hb1, mb0, mb1,
              semA0, semA1, semG0, semG1, semS0, semS1):
    c = lax.axis_index("c")
    s = lax.axis_index("s")
    hgs = (hg0, hg1, hg2, hg3)
    rwp = (rwA, rwB)
    aggs = (agg0, agg1, agg2, agg3)
    sidx = (sidx0, sidx1)
    ridx = (ridx0, ridx1)
    rs = (rs0, rs1)
    shb = (shb0, shb1)
    rwb = (rwb0, rwb1)
    hb = (hb0, hb1)
    mb = (mb0, mb1)
    semA = (semA0, semA1)
    semG = (semG0, semG1)
    semS = (semS0, semS1)
    zero16 = jnp.zeros((16,), _F32)
    c1 = _full16(1)
    c2 = _full16(2)
    c3 = _full16(3)

    def _a_copies(j, sl, rwg):
        src, off = rwg
        e0 = s * MCH + j * NCHUNK
        return (
            (send_hbm.at[pl.ds(e0, NCHUNK)], sidx[sl]),
            (recv_hbm.at[pl.ds(e0, NCHUNK)], ridx[sl]),
            (sh_hbm.at[pl.ds(e0 * 4, NCHUNK * 4)], shb[sl].at[pl.ds(0, NCHUNK * 4)]),
            (src.at[pl.ds(e0, NCHUNK), pl.ds(off, 64)], rwb[sl]),
        )

    def _issue_a(j, sl, rwg):
        for src, dst in _a_copies(j, sl, rwg):
            pltpu.async_copy(src, dst, semA[sl])

    def _wait_a(j, sl, rwg):
        for src, dst in _a_copies(j, sl, rwg):
            pltpu.make_async_copy(src, dst, semA[sl]).wait()

    def _issue_g(sl, hg):
        pltpu.async_copy(hg.at[sidx[sl]], hb[sl], semG[sl])

    def _wait_g(sl, hg):
        pltpu.make_async_copy(hg.at[sidx[sl]], hb[sl], semG[sl]).wait()

    def _copy_ridx(sl):
        for q in range(NCHUNK // 16):
            rs[sl][pl.ds(q * 16, 16)] = ridx[sl][pl.ds(q * 16, 16)]

    def _issue_s(sl):
        pltpu.async_copy(mb[sl], acc.at[rs[sl]], semS[sl], add=True)

    def _wait_s(sl):
        pltpu.make_async_copy(mb[sl], acc.at[rs[sl]], semS[sl]).wait()

    def _compute(sl):
        hbuf, rwbuf, shbuf = hb[sl], rwb[sl], shb[sl]
        msgbuf = mb[sl]

        @pl.loop(0, NCHUNK)
        def _edge(i):
            ha = hbuf[i, 0:16]
            hb_ = hbuf[i, 16:32]
            r0a = rwbuf[i, 0:16]
            r0b = rwbuf[i, 16:32]
            r1a = rwbuf[i, 32:48]
            r1b = rwbuf[i, 48:64]
            sv = shbuf[pl.ds(i * 4, 16)]
            s1 = sv[c1]
            s2 = sv[c2]
            s3 = sv[c3]
            h1a = ha * r1a
            h1b = hb_ * r1b
            msgbuf[i, 0:16] = ha * r0a
            msgbuf[i, 16:32] = hb_ * r0b
            msgbuf[i, 32:48] = h1a * s1
            msgbuf[i, 48:64] = h1b * s1
            msgbuf[i, 64:80] = h1a * s2
            msgbuf[i, 80:96] = h1b * s2
            msgbuf[i, 96:112] = h1a * s3
            msgbuf[i, 112:128] = h1b * s3

    def _pipeline(hg, rwg):
        _issue_a(0, 0, rwg)
        _wait_a(0, 0, rwg)
        _copy_ridx(0)
        _issue_g(0, hg)
        _issue_a(1, 1, rwg)

        @pl.loop(0, NCH // 2)
        def _chunk(jj):
            for half in range(2):
                sl, o = half, 1 - half
                j = jj * 2 + half

                @pl.when(j + 1 < NCH)
                def _():
                    @pl.when(j >= 1)
                    def _():
                        _wait_s(o)

                    _wait_a(j + 1, o, rwg)
                    _copy_ridx(o)
                    _issue_g(o, hg)

                _wait_g(sl, hg)
                _compute(sl)
                _issue_s(sl)

                @pl.when(j + 2 < NCH)
                def _():
                    _issue_a(j + 2, sl, rwg)

        _wait_s(0)
        _wait_s(1)

    for p in range(2):
        @pl.loop(0, NCHUNK)
        def _z(i):
            for jz in range(8):
                mb0[i, 16 * jz:16 * (jz + 1)] = zero16

        for z in range(NACC // (NSUB * NCHUNK)):
            pltpu.sync_copy(mb0, acc.at[pl.ds((s * (NACC // (NSUB * NCHUNK)) + z) * NCHUNK, NCHUNK)])
        plsc.subcore_barrier()

        for cs in range(NCORES):
            g = NCORES * cs + p

            @pl.when(c == cs)
            def _():
                _pipeline(hgs[g], (rwp[p], 64 * cs))

        plsc.subcore_barrier()
        for cs in range(NCORES):
            g = NCORES * cs + p

            @pl.when(c == cs)
            def _():
                pltpu.sync_copy(acc.at[pl.ds(s * NWB, NWB)],
                                aggs[g].at[pl.ds(s * NWB, NWB)])
        plsc.subcore_barrier()


def _message(send_p, recv_p, hgs, rwpair, sh_p):
    mesh = plsc.VectorSubcoreMesh(core_axis_name="c", subcore_axis_name="s")
    return pl.kernel(
        _msg_body,
        out_type=[jax.ShapeDtypeStruct((NACC, C), _F32)] * 4,
        mesh=mesh,
        compiler_params=pltpu.CompilerParams(needs_layout_passes=False,
                                             use_tc_tiling_on_sc=False),
        scratch_types=[
            pltpu.MemorySpace.VMEM_SHARED((NACC, C), _F32),
            pltpu.MemorySpace.VMEM((NCHUNK,), _I32),
            pltpu.MemorySpace.VMEM((NCHUNK,), _I32),
            pltpu.MemorySpace.VMEM((NCHUNK,), _I32),
            pltpu.MemorySpace.VMEM((NCHUNK,), _I32),
            pltpu.MemorySpace.VMEM((NCHUNK,), _I32),
            pltpu.MemorySpace.VMEM((NCHUNK,), _I32),
            pltpu.MemorySpace.VMEM((NCHUNK * 4 + 16,), _F32),
            pltpu.MemorySpace.VMEM((NCHUNK * 4 + 16,), _F32),
            pltpu.MemorySpace.VMEM((NCHUNK, 64), _F32),
            pltpu.MemorySpace.VMEM((NCHUNK, 64), _F32),
            pltpu.MemorySpace.VMEM((NCHUNK, 32), _F32),
            pltpu.MemorySpace.VMEM((NCHUNK, 32), _F32),
            pltpu.MemorySpace.VMEM((NCHUNK, C), _F32),
            pltpu.MemorySpace.VMEM((NCHUNK, C), _F32),
            pltpu.SemaphoreType.DMA,
            pltpu.SemaphoreType.DMA,
            pltpu.SemaphoreType.DMA,
            pltpu.SemaphoreType.DMA,
            pltpu.SemaphoreType.DMA,
            pltpu.SemaphoreType.DMA,
        ],
    )(send_p, recv_p, *hgs, *rwpair, sh_p)


def _edge_body(vec_ref, w1a_ref, w2a_ref, w3a_ref, w1b_ref, w2b_ref, w3b_ref,
               rwA0, rwB0, rwA1, rwB1, sh_ref):
    vec = vec_ref[...]
    d2 = jnp.sum(vec * vec, axis=1, keepdims=True) + 1e-12
    r = jnp.sqrt(d2)
    inv_r = 1.0 / r
    unit = vec * inv_r
    sh_ref[...] = jnp.concatenate(
        [jnp.ones((vec.shape[0], 1), _F32), np.sqrt(3.0).astype(np.float32) * unit], axis=1)
    n = (lax.broadcasted_iota(_I32, (vec.shape[0], NB), 1).astype(_F32)
         + 1.0) * (np.pi / RMAX)
    arg = r * n
    u = r * (1.0 / RMAX)
    u5 = u * u * u * u * u
    env = 1.0 - 21.0 * u5 + 35.0 * u5 * u - 15.0 * u5 * u * u
    env = jnp.where(u < 1.0, env, 0.0)
    pref = np.sqrt(2.0 / RMAX).astype(np.float32)
    ef = (pref * jnp.sin(arg)) * (inv_r * env)
    for w1, w2, w3, outs in ((w1a_ref, w2a_ref, w3a_ref, (rwA0, rwB0)),
                             (w1b_ref, w2b_ref, w3b_ref, (rwA1, rwB1))):
        t = C2M * _silu(jnp.dot(ef, w1[...], preferred_element_type=_F32))
        t = C2M * _silu(jnp.dot(t, w2[...], preferred_element_type=_F32))
        full = jnp.dot(t, w3[...], preferred_element_type=_F32)
        outs[0][...] = full[:, 0:128]
        outs[1][...] = full[:, 128:256]


def _edge_pass(vec, w3p_0, w3p_1, rW1_0, rW2_0, rW1_1, rW2_1):
    return pl.pallas_call(
        _edge_body,
        grid=(EP // BE,),
        in_specs=[
            pl.BlockSpec((BE, 3), lambda i: (i, 0)),
            pl.BlockSpec((NB, 64), lambda i: (0, 0)),
            pl.BlockSpec((64, 64), lambda i: (0, 0)),
            pl.BlockSpec((64, 2 * C), lambda i: (0, 0)),
            pl.BlockSpec((NB, 64), lambda i: (0, 0)),
            pl.BlockSpec((64, 64), lambda i: (0, 0)),
            pl.BlockSpec((64, 2 * C), lambda i: (0, 0)),
        ],
        out_specs=[pl.BlockSpec((BE, 2 * 64), lambda i: (i, 0))] * 4
        + [pl.BlockSpec((BE, 4), lambda i: (i, 0))],
        out_shape=[jax.ShapeDtypeStruct((EP, 2 * 64), _F32)] * 4
        + [jax.ShapeDtypeStruct((EP, 4), _F32)],
    )(vec, rW1_0, rW2_0, w3p_0, rW1_1, rW2_1, w3p_1)


def _embed_body(na_ref, wemb_ref, e0w_ref, hg0, hg1, hg2, hg3, e_ref):
    na = na_ref[...]
    h = jnp.dot(na, wemb_ref[...], preferred_element_type=_F32)
    for g in range(4):
        (hg0, hg1, hg2, hg3)[g][...] = h[:, 32 * g:32 * (g + 1)]
    e_ref[...] = jnp.dot(na, e0w_ref[...], preferred_element_type=_F32)


def _embed(node_attrs, W_emb, E0_w):
    return pl.pallas_call(
        _embed_body,
        grid=(N // BN,),
        in_specs=[
            pl.BlockSpec((BN, NE), lambda i: (i, 0)),
            pl.BlockSpec((NE, C), lambda i: (0, 0)),
            pl.BlockSpec((NE, 1), lambda i: (0, 0)),
        ],
        out_specs=[pl.BlockSpec((BN, 32), lambda i: (i, 0))] * 4
        + [pl.BlockSpec((BN, 1), lambda i: (i, 0))],
        out_shape=[jax.ShapeDtypeStruct((N, 32), _F32)] * 4
        + [jax.ShapeDtypeStruct((N, 1), _F32)],
    )(node_attrs, W_emb, E0_w.reshape(NE, 1))


def _inv_from_agg(agg_refs):
    pieces = []
    for g in range(4):
        a = agg_refs[g][...]
        a0 = a[:, 0:32] * (1.0 / AVG)
        sq = (a[:, 32:64] ** 2 + a[:, 64:96] ** 2 + a[:, 96:128] ** 2) * (1.0 / (AVG * AVG))
        pieces.append(a0 + sq)
    return jnp.concatenate(pieces, axis=1)


def _node0_body(a0, a1, a2, a3, wm_ref, wro_ref, hg0, hg1, hg2, hg3, e_ref):
    inv = _inv_from_agg((a0, a1, a2, a3))
    h = jnp.dot(inv, wm_ref[...], preferred_element_type=_F32)
    for g in range(4):
        (hg0, hg1, hg2, hg3)[g][...] = h[:, 32 * g:32 * (g + 1)]
    e_ref[...] = jnp.dot(h, wro_ref[...], preferred_element_type=_F32)


def _node0(aggs, Wmix, Wro):
    return pl.pallas_call(
        _node0_body,
        grid=(N // BN,),
        in_specs=[pl.BlockSpec((BN, C), lambda i: (i, 0))] * 4
        + [pl.BlockSpec((C, C), lambda i: (0, 0)),
           pl.BlockSpec((C, 1), lambda i: (0, 0))],
        out_specs=[pl.BlockSpec((BN, 32), lambda i: (i, 0))] * 4
        + [pl.BlockSpec((BN, 1), lambda i: (i, 0))],
        out_shape=[jax.ShapeDtypeStruct((N, 32), _F32)] * 4
        + [jax.ShapeDtypeStruct((N, 1), _F32)],
    )(*aggs, Wmix, Wro)


def _node1_body(a0, a1, a2, a3, wm_ref, wh_ref, wo_ref, e_ref):
    inv = _inv_from_agg((a0, a1, a2, a3))
    h = jnp.dot(inv, wm_ref[...], preferred_element_type=_F32)
    hh = C2M * _silu(jnp.dot(h, wh_ref[...], preferred_element_type=_F32))
    e_ref[...] = jnp.dot(hh, wo_ref[...], preferred_element_type=_F32)


def _node1(aggs, Wmix, Wh, Wo):
    return pl.pallas_call(
        _node1_body,
        grid=(N // BN,),
        in_specs=[pl.BlockSpec((BN, C), lambda i: (i, 0))] * 4
        + [pl.BlockSpec((C, C), lambda i: (0, 0)),
           pl.BlockSpec((C, 16), lambda i: (0, 0)),
           pl.BlockSpec((16, 1), lambda i: (0, 0))],
        out_specs=pl.BlockSpec((BN, 1), lambda i: (i, 0)),
        out_shape=jax.ShapeDtypeStruct((N, 1), _F32),
    )(*aggs, Wmix, Wh, Wo)


def kernel(node_attrs, positions, shifts, W_emb, E0_w,
           rW1_0, rW2_0, rW3_0, Wmix_0, Wro_0,
           rW1_1, rW2_1, rW3_1, Wmix_1, Wh, Wo, edge_index):
    sender = edge_index[0].astype(_I32)
    receiver = edge_index[1].astype(_I32)
    pad = EP - E
    send_p = jnp.concatenate([sender, jnp.zeros((pad,), _I32)])
    recv_g = jnp.concatenate([receiver, jnp.zeros((pad,), _I32)])
    recv_m = jnp.concatenate([receiver, jnp.full((pad,), DUMMY, _I32)])

    perm = np.array([(32 * g + cp) * 2 + path
                     for g in (0, 2, 1, 3) for path in range(2) for cp in range(32)])
    w3p_0 = rW3_0[:, perm]
    w3p_1 = rW3_1[:, perm]

    vec = _geometry(positions.reshape(N * 3), send_p, recv_g).reshape(EP, 3)

    eouts = _edge_pass(vec, w3p_0, w3p_1, rW1_0, rW2_0, rW1_1, rW2_1)
    rwg0, rwg1, sh_p = eouts[0:2], eouts[2:4], eouts[4].reshape(EP * 4)

    *hgs, e0 = _embed(node_attrs, W_emb, E0_w)
    e = e0[:, 0]

    aggs = _message(send_p, recv_m, hgs, rwg0, sh_p)
    *hgs, ep1 = _node0(aggs, Wmix_0, Wro_0)
    e = e + ep1[:, 0]

    aggs = _message(send_p, recv_m, hgs, rwg1, sh_p)
    e = e + _node1(aggs, Wmix_1, Wh, Wo)[:, 0]
    return e


# speedup vs baseline: 1.7578x; 1.0327x over previous
import functools

import jax
import jax.numpy as jnp
import numpy as np
from jax import lax
from jax.experimental import pallas as pl
from jax.experimental.pallas import tpu as pltpu
from jax.experimental.pallas import tpu_sc as plsc

N = 10000
E = 160000
NE = 4
C = 128
RMAX = 5.0
NB = 8
P = 5
AVG = 16.0
C2M = 1.6792

NCORES = 2
NSUB = 16
EP = 163840
GCH = EP // (NCORES * NSUB)
MCH = EP // NSUB
NCHUNK = 64
NACC = 10240
DUMMY = N
NWB = NACC // NSUB

BE = 2048
BN = 2000

_I32 = jnp.int32
_F32 = jnp.float32


def _silu(x):
    return x * jax.nn.sigmoid(x)


def _full16(v):
    return jnp.full((16,), v, _I32)


def _geom_body(pos_hbm, send_hbm, recv_hbm, vec_hbm,
               posb, sbuf, rbuf, vb):
    c = lax.axis_index("c")
    s = lax.axis_index("s")
    wid = s * NCORES + c
    base = wid * GCH
    pltpu.sync_copy(pos_hbm, posb)
    pltpu.sync_copy(send_hbm.at[pl.ds(base, GCH)], sbuf)
    pltpu.sync_copy(recv_hbm.at[pl.ds(base, GCH)], rbuf)
    iota16 = lax.iota(_I32, 16)
    three = _full16(3)

    @pl.loop(0, GCH // 16)
    def _micro(m):
        off = pl.multiple_of(m * 16, 16)
        sidx = sbuf[pl.ds(off, 16)] * three
        ridx = rbuf[pl.ds(off, 16)] * three
        lidx = (jnp.full((16,), off, _I32) + iota16) * three
        for k in range(3):
            kc = _full16(k)
            p_s = plsc.load_gather(posb, [sidx + kc])
            p_r = plsc.load_gather(posb, [ridx + kc])
            plsc.store_scatter(vb, [lidx + kc], p_r - p_s)

    pltpu.sync_copy(vb, vec_hbm.at[pl.ds(base * 3, GCH * 3)])


def _geometry(positions, send_p, recv_p):
    mesh = plsc.VectorSubcoreMesh(core_axis_name="c", subcore_axis_name="s")
    return pl.kernel(
        _geom_body,
        out_type=jax.ShapeDtypeStruct((EP * 3,), _F32),
        mesh=mesh,
        compiler_params=pltpu.CompilerParams(needs_layout_passes=False),
        scratch_types=[
            pltpu.MemorySpace.VMEM((N * 3,), _F32),
            pltpu.MemorySpace.VMEM((GCH,), _I32),
            pltpu.MemorySpace.VMEM((GCH,), _I32),
            pltpu.MemorySpace.VMEM((GCH * 3,), _F32),
        ],
    )(positions, send_p, recv_p)


NCH = MCH // NCHUNK


def _msg_body(send_hbm, recv_hbm, hg0, hg1, hg2, hg3, rwA, rwB,
              sh_hbm, agg0, agg1, agg2, agg3,
              acc, sidx0, sidx1, ridx0, ridx1, rs0, rs1, shb0, shb1,
              rwb0, rwb1, hb0, hb1, mb0, mb1,
              semA0, semA1, semG0, semG1, semS0, semS1):
    c = lax.axis_index("c")
    s = lax.axis_index("s")
    hgs = (hg0, hg1, hg2, hg3)
    rwp = (rwA, rwB)
    aggs = (agg0, agg1, agg2, agg3)
    sidx = (sidx0, sidx1)
    ridx = (ridx0, ridx1)
    rs = (rs0, rs1)
    shb = (shb0, shb1)
    rwb = (rwb0, rwb1)
    hb = (hb0, hb1)
    mb = (mb0, mb1)
    semA = (semA0, semA1)
    semG = (semG0, semG1)
    semS = (semS0, semS1)
    zero16 = jnp.zeros((16,), _F32)
    c1 = _full16(1)
    c2 = _full16(2)
    c3 = _full16(3)

    def _a_copies(j, sl, rwg):
        src, off = rwg
        e0 = s * MCH + j * NCHUNK
        return (
            (send_hbm.at[pl.ds(e0, NCHUNK)], sidx[sl]),
            (recv_hbm.at[pl.ds(e0, NCHUNK)], ridx[sl]),
            (sh_hbm.at[pl.ds(e0 * 4, NCHUNK * 4)], shb[sl].at[pl.ds(0, NCHUNK * 4)]),
            (src.at[pl.ds(e0, NCHUNK), pl.ds(off, 64)], rwb[sl]),
        )

    def _issue_a(j, sl, rwg):
        for src, dst in _a_copies(j, sl, rwg):
            pltpu.async_copy(src, dst, semA[sl])

    def _wait_a(j, sl, rwg):
        for src, dst in _a_copies(j, sl, rwg):
            pltpu.make_async_copy(src, dst, semA[sl]).wait()

    def _issue_g(sl, hg):
        pltpu.async_copy(hg.at[sidx[sl]], hb[sl], semG[sl])

    def _wait_g(sl, hg):
        pltpu.make_async_copy(hg.at[sidx[sl]], hb[sl], semG[sl]).wait()

    def _copy_ridx(sl):
        for q in range(NCHUNK // 16):
            rs[sl][pl.ds(q * 16, 16)] = ridx[sl][pl.ds(q * 16, 16)]

    def _issue_s(sl):
        pltpu.async_copy(mb[sl], acc.at[rs[sl]], semS[sl], add=True)

    def _wait_s(sl):
        pltpu.make_async_copy(mb[sl], acc.at[rs[sl]], semS[sl]).wait()

    def _compute(sl):
        hbuf, rwbuf, shbuf = hb[sl], rwb[sl], shb[sl]
        msgbuf = mb[sl]

        @pl.loop(0, NCHUNK)
        def _edge(i):
            ha = hbuf[i, 0:16]
            hb_ = hbuf[i, 16:32]
            r0a = rwbuf[i, 0:16]
            r0b = rwbuf[i, 16:32]
            r1a = rwbuf[i, 32:48]
            r1b = rwbuf[i, 48:64]
            sv = shbuf[pl.ds(i * 4, 16)]
            s1 = sv[c1]
            s2 = sv[c2]
            s3 = sv[c3]
            h1a = ha * r1a
            h1b = hb_ * r1b
            msgbuf[i, 0:16] = ha * r0a
            msgbuf[i, 16:32] = hb_ * r0b
            msgbuf[i, 32:48] = h1a * s1
            msgbuf[i, 48:64] = h1b * s1
            msgbuf[i, 64:80] = h1a * s2
            msgbuf[i, 80:96] = h1b * s2
            msgbuf[i, 96:112] = h1a * s3
            msgbuf[i, 112:128] = h1b * s3

    def _pipeline(hg, rwg):
        _issue_a(0, 0, rwg)
        _wait_a(0, 0, rwg)
        _copy_ridx(0)
        _issue_g(0, hg)
        _issue_a(1, 1, rwg)

        @pl.loop(0, NCH // 2)
        def _chunk(jj):
            for half in range(2):
                sl, o = half, 1 - half
                j = jj * 2 + half

                @pl.when(j + 1 < NCH)
                def _():
                    @pl.when(j >= 1)
                    def _():
                        _wait_s(o)

                    _wait_a(j + 1, o, rwg)
                    _copy_ridx(o)
                    _issue_g(o, hg)

                _wait_g(sl, hg)
                _compute(sl)
                _issue_s(sl)

                @pl.when(j + 2 < NCH)
                def _():
                    _issue_a(j + 2, sl, rwg)

        _wait_s(0)
        _wait_s(1)

    for p in range(2):
        @pl.loop(0, NCHUNK)
        def _z(i):
            for jz in range(8):
                mb0[i, 16 * jz:16 * (jz + 1)] = zero16

        for z in range(NACC // (NSUB * NCHUNK)):
            pltpu.sync_copy(mb0, acc.at[pl.ds((s * (NACC // (NSUB * NCHUNK)) + z) * NCHUNK, NCHUNK)])
        plsc.subcore_barrier()

        for cs in range(NCORES):
            g = NCORES * cs + p

            @pl.when(c == cs)
            def _():
                _pipeline(hgs[g], (rwp[p], 64 * cs))

        plsc.subcore_barrier()
        for cs in range(NCORES):
            g = NCORES * cs + p

            @pl.when(c == cs)
            def _():
                pltpu.sync_copy(acc.at[pl.ds(s * NWB, NWB)],
                                aggs[g].at[pl.ds(s * NWB, NWB)])
        plsc.subcore_barrier()


def _message(send_p, recv_p, hgs, rwpair, sh_p):
    mesh = plsc.VectorSubcoreMesh(core_axis_name="c", subcore_axis_name="s")
    return pl.kernel(
        _msg_body,
        out_type=[jax.ShapeDtypeStruct((NACC, C), _F32)] * 4,
        mesh=mesh,
        compiler_params=pltpu.CompilerParams(needs_layout_passes=False,
                                             use_tc_tiling_on_sc=False),
        scratch_types=[
            pltpu.MemorySpace.VMEM_SHARED((NACC, C), _F32),
            pltpu.MemorySpace.VMEM((NCHUNK,), _I32),
            pltpu.MemorySpace.VMEM((NCHUNK,), _I32),
            pltpu.MemorySpace.VMEM((NCHUNK,), _I32),
            pltpu.MemorySpace.VMEM((NCHUNK,), _I32),
            pltpu.MemorySpace.VMEM((NCHUNK,), _I32),
            pltpu.MemorySpace.VMEM((NCHUNK,), _I32),
            pltpu.MemorySpace.VMEM((NCHUNK * 4 + 16,), _F32),
            pltpu.MemorySpace.VMEM((NCHUNK * 4 + 16,), _F32),
            pltpu.MemorySpace.VMEM((NCHUNK, 64), _F32),
            pltpu.MemorySpace.VMEM((NCHUNK, 64), _F32),
            pltpu.MemorySpace.VMEM((NCHUNK, 32), _F32),
            pltpu.MemorySpace.VMEM((NCHUNK, 32), _F32),
            pltpu.MemorySpace.VMEM((NCHUNK, C), _F32),
            pltpu.MemorySpace.VMEM((NCHUNK, C), _F32),
            pltpu.SemaphoreType.DMA,
            pltpu.SemaphoreType.DMA,
            pltpu.SemaphoreType.DMA,
            pltpu.SemaphoreType.DMA,
            pltpu.SemaphoreType.DMA,
            pltpu.SemaphoreType.DMA,
        ],
    )(send_p, recv_p, *hgs, *rwpair, sh_p)


def _edge_body(vec_ref, w1_ref, w2_ref, w3a_ref, w3b_ref,
               rwA0, rwB0, rwA1, rwB1, sh_ref):
    vec = vec_ref[...]
    d2 = jnp.sum(vec * vec, axis=1, keepdims=True) + 1e-12
    r = jnp.sqrt(d2)
    inv_r = 1.0 / r
    unit = vec * inv_r
    sh_ref[...] = jnp.concatenate(
        [jnp.ones((vec.shape[0], 1), _F32), np.sqrt(3.0).astype(np.float32) * unit], axis=1)
    n = (lax.broadcasted_iota(_I32, (vec.shape[0], NB), 1).astype(_F32)
         + 1.0) * (np.pi / RMAX)
    arg = r * n
    u = r * (1.0 / RMAX)
    u5 = u * u * u * u * u
    env = 1.0 - 21.0 * u5 + 35.0 * u5 * u - 15.0 * u5 * u * u
    env = jnp.where(u < 1.0, env, 0.0)
    pref = np.sqrt(2.0 / RMAX).astype(np.float32)
    ef = (pref * jnp.sin(arg)) * (inv_r * env)
    bf = jnp.bfloat16
    t = C2M * _silu(jnp.dot(ef.astype(bf), w1_ref[...],
                            preferred_element_type=_F32))
    t = C2M * _silu(jnp.dot(t.astype(bf), w2_ref[...],
                            preferred_element_type=_F32))
    t16 = t.astype(bf)
    full0 = jnp.dot(t16[:, 0:64], w3a_ref[...], preferred_element_type=_F32)
    full1 = jnp.dot(t16[:, 64:128], w3b_ref[...], preferred_element_type=_F32)
    rwA0[...] = full0[:, 0:128]
    rwB0[...] = full0[:, 128:256]
    rwA1[...] = full1[:, 0:128]
    rwB1[...] = full1[:, 128:256]


def _edge_pass(vec, w1c, w2b, w3a, w3b):
    return pl.pallas_call(
        _edge_body,
        grid=(EP // BE,),
        in_specs=[
            pl.BlockSpec((BE, 3), lambda i: (i, 0)),
            pl.BlockSpec((NB, 2 * 64), lambda i: (0, 0)),
            pl.BlockSpec((2 * 64, 2 * 64), lambda i: (0, 0)),
            pl.BlockSpec((64, 2 * C), lambda i: (0, 0)),
            pl.BlockSpec((64, 2 * C), lambda i: (0, 0)),
        ],
        out_specs=[pl.BlockSpec((BE, 2 * 64), lambda i: (i, 0))] * 4
        + [pl.BlockSpec((BE, 4), lambda i: (i, 0))],
        out_shape=[jax.ShapeDtypeStruct((EP, 2 * 64), _F32)] * 4
        + [jax.ShapeDtypeStruct((EP, 4), _F32)],
    )(vec, w1c, w2b, w3a, w3b)


def _embed_body(na_ref, wemb_ref, e0w_ref, hg0, hg1, hg2, hg3, e_ref):
    na = na_ref[...]
    h = jnp.dot(na, wemb_ref[...], preferred_element_type=_F32)
    for g in range(4):
        (hg0, hg1, hg2, hg3)[g][...] = h[:, 32 * g:32 * (g + 1)]
    e_ref[...] = jnp.dot(na, e0w_ref[...], preferred_element_type=_F32)


def _embed(node_attrs, W_emb, E0_w):
    return pl.pallas_call(
        _embed_body,
        grid=(N // BN,),
        in_specs=[
            pl.BlockSpec((BN, NE), lambda i: (i, 0)),
            pl.BlockSpec((NE, C), lambda i: (0, 0)),
            pl.BlockSpec((NE, 1), lambda i: (0, 0)),
        ],
        out_specs=[pl.BlockSpec((BN, 32), lambda i: (i, 0))] * 4
        + [pl.BlockSpec((BN, 1), lambda i: (i, 0))],
        out_shape=[jax.ShapeDtypeStruct((N, 32), _F32)] * 4
        + [jax.ShapeDtypeStruct((N, 1), _F32)],
    )(node_attrs, W_emb, E0_w.reshape(NE, 1))


def _inv_from_agg(agg_refs):
    pieces = []
    for g in range(4):
        a = agg_refs[g][...]
        a0 = a[:, 0:32] * (1.0 / AVG)
        sq = (a[:, 32:64] ** 2 + a[:, 64:96] ** 2 + a[:, 96:128] ** 2) * (1.0 / (AVG * AVG))
        pieces.append(a0 + sq)
    return jnp.concatenate(pieces, axis=1)


def _node0_body(a0, a1, a2, a3, wm_ref, wro_ref, hg0, hg1, hg2, hg3, e_ref):
    inv = _inv_from_agg((a0, a1, a2, a3))
    h = jnp.dot(inv, wm_ref[...], preferred_element_type=_F32)
    for g in range(4):
        (hg0, hg1, hg2, hg3)[g][...] = h[:, 32 * g:32 * (g + 1)]
    e_ref[...] = jnp.dot(h, wro_ref[...], preferred_element_type=_F32)


def _node0(aggs, Wmix, Wro):
    return pl.pallas_call(
        _node0_body,
        grid=(N // BN,),
        in_specs=[pl.BlockSpec((BN, C), lambda i: (i, 0))] * 4
        + [pl.BlockSpec((C, C), lambda i: (0, 0)),
           pl.BlockSpec((C, 1), lambda i: (0, 0))],
        out_specs=[pl.BlockSpec((BN, 32), lambda i: (i, 0))] * 4
        + [pl.BlockSpec((BN, 1), lambda i: (i, 0))],
        out_shape=[jax.ShapeDtypeStruct((N, 32), _F32)] * 4
        + [jax.ShapeDtypeStruct((N, 1), _F32)],
    )(*aggs, Wmix, Wro)


def _node1_body(a0, a1, a2, a3, wm_ref, wh_ref, wo_ref, e_ref):
    inv = _inv_from_agg((a0, a1, a2, a3))
    h = jnp.dot(inv, wm_ref[...], preferred_element_type=_F32)
    hh = C2M * _silu(jnp.dot(h, wh_ref[...], preferred_element_type=_F32))
    e_ref[...] = jnp.dot(hh, wo_ref[...], preferred_element_type=_F32)


def _node1(aggs, Wmix, Wh, Wo):
    return pl.pallas_call(
        _node1_body,
        grid=(N // BN,),
        in_specs=[pl.BlockSpec((BN, C), lambda i: (i, 0))] * 4
        + [pl.BlockSpec((C, C), lambda i: (0, 0)),
           pl.BlockSpec((C, 16), lambda i: (0, 0)),
           pl.BlockSpec((16, 1), lambda i: (0, 0))],
        out_specs=pl.BlockSpec((BN, 1), lambda i: (i, 0)),
        out_shape=jax.ShapeDtypeStruct((N, 1), _F32),
    )(*aggs, Wmix, Wh, Wo)


def kernel(node_attrs, positions, shifts, W_emb, E0_w,
           rW1_0, rW2_0, rW3_0, Wmix_0, Wro_0,
           rW1_1, rW2_1, rW3_1, Wmix_1, Wh, Wo, edge_index):
    sender = edge_index[0].astype(_I32)
    receiver = edge_index[1].astype(_I32)
    pad = EP - E
    send_p = jnp.concatenate([sender, jnp.zeros((pad,), _I32)])
    recv_g = jnp.concatenate([receiver, jnp.zeros((pad,), _I32)])
    recv_m = jnp.concatenate([receiver, jnp.full((pad,), DUMMY, _I32)])

    perm = np.array([(32 * g + cp) * 2 + path
                     for g in (0, 2, 1, 3) for path in range(2) for cp in range(32)])
    bf = jnp.bfloat16
    w3a = rW3_0[:, perm].astype(bf)
    w3b = rW3_1[:, perm].astype(bf)
    w1c = jnp.concatenate([rW1_0, rW1_1], axis=1).astype(bf)
    w2b = jnp.zeros((128, 128), _F32)
    w2b = w2b.at[0:64, 0:64].set(rW2_0).at[64:128, 64:128].set(rW2_1).astype(bf)

    vec = _geometry(positions.reshape(N * 3), send_p, recv_g).reshape(EP, 3)

    eouts = _edge_pass(vec, w1c, w2b, w3a, w3b)
    rwg0, rwg1, sh_p = eouts[0:2], eouts[2:4], eouts[4].reshape(EP * 4)

    *hgs, e0 = _embed(node_attrs, W_emb, E0_w)
    e = e0[:, 0]

    aggs = _message(send_p, recv_m, hgs, rwg0, sh_p)
    *hgs, ep1 = _node0(aggs, Wmix_0, Wro_0)
    e = e + ep1[:, 0]

    aggs = _message(send_p, recv_m, hgs, rwg1, sh_p)
    e = e + _node1(aggs, Wmix_1, Wh, Wo)[:, 0]
    return e


# speedup vs baseline: 1.8228x; 1.0369x over previous
import functools

import jax
import jax.numpy as jnp
import numpy as np
from jax import lax
from jax.experimental import pallas as pl
from jax.experimental.pallas import tpu as pltpu
from jax.experimental.pallas import tpu_sc as plsc

N = 10000
E = 160000
NE = 4
C = 128
RMAX = 5.0
NB = 8
P = 5
AVG = 16.0
C2M = 1.6792

NCORES = 2
NSUB = 16
EP = 163840
GCH = EP // (NCORES * NSUB)
MCH = EP // NSUB
NCHUNK = 80
NACC = 10240
DUMMY = N
NWB = NACC // NSUB

BE = 2048
BN = 2000

_I32 = jnp.int32
_F32 = jnp.float32


def _silu(x):
    return x * jax.nn.sigmoid(x)


def _full16(v):
    return jnp.full((16,), v, _I32)


def _geom_body(pos_hbm, send_hbm, recv_hbm, vec_hbm,
               posb, sbuf, rbuf, vb):
    c = lax.axis_index("c")
    s = lax.axis_index("s")
    wid = s * NCORES + c
    base = wid * GCH
    pltpu.sync_copy(pos_hbm, posb)
    pltpu.sync_copy(send_hbm.at[pl.ds(base, GCH)], sbuf)
    pltpu.sync_copy(recv_hbm.at[pl.ds(base, GCH)], rbuf)
    iota16 = lax.iota(_I32, 16)
    three = _full16(3)

    @pl.loop(0, GCH // 16)
    def _micro(m):
        off = pl.multiple_of(m * 16, 16)
        sidx = sbuf[pl.ds(off, 16)] * three
        ridx = rbuf[pl.ds(off, 16)] * three
        lidx = (jnp.full((16,), off, _I32) + iota16) * three
        for k in range(3):
            kc = _full16(k)
            p_s = plsc.load_gather(posb, [sidx + kc])
            p_r = plsc.load_gather(posb, [ridx + kc])
            plsc.store_scatter(vb, [lidx + kc], p_r - p_s)

    pltpu.sync_copy(vb, vec_hbm.at[pl.ds(base * 3, GCH * 3)])


def _geometry(positions, send_p, recv_p):
    mesh = plsc.VectorSubcoreMesh(core_axis_name="c", subcore_axis_name="s")
    return pl.kernel(
        _geom_body,
        out_type=jax.ShapeDtypeStruct((EP * 3,), _F32),
        mesh=mesh,
        compiler_params=pltpu.CompilerParams(needs_layout_passes=False),
        scratch_types=[
            pltpu.MemorySpace.VMEM((N * 3,), _F32),
            pltpu.MemorySpace.VMEM((GCH,), _I32),
            pltpu.MemorySpace.VMEM((GCH,), _I32),
            pltpu.MemorySpace.VMEM((GCH * 3,), _F32),
        ],
    )(positions, send_p, recv_p)


NCH = MCH // NCHUNK


def _msg_body(send_hbm, recv_hbm, hg0, hg1, hg2, hg3, rwA, rwB,
              sh_hbm, agg0, agg1, agg2, agg3,
              acc, sidx0, sidx1, ridx0, ridx1, rs0, rs1, shb0, shb1,
              rwb0, rwb1, hb0, hb1, mb0, mb1,
              semA0, semA1, semG0, semG1, semS0, semS1):
    c = lax.axis_index("c")
    s = lax.axis_index("s")
    hgs = (hg0, hg1, hg2, hg3)
    rwp = (rwA, rwB)
    aggs = (agg0, agg1, agg2, agg3)
    sidx = (sidx0, sidx1)
    ridx = (ridx0, ridx1)
    rs = (rs0, rs1)
    shb = (shb0, shb1)
    rwb = (rwb0, rwb1)
    hb = (hb0, hb1)
    mb = (mb0, mb1)
    semA = (semA0, semA1)
    semG = (semG0, semG1)
    semS = (semS0, semS1)
    zero16 = jnp.zeros((16,), _F32)
    c1 = _full16(1)
    c2 = _full16(2)
    c3 = _full16(3)

    def _a_copies(j, sl, rwg):
        src, off = rwg
        e0 = s * MCH + j * NCHUNK
        return (
            (send_hbm.at[pl.ds(e0, NCHUNK)], sidx[sl]),
            (recv_hbm.at[pl.ds(e0, NCHUNK)], ridx[sl]),
            (sh_hbm.at[pl.ds(e0 * 4, NCHUNK * 4)], shb[sl].at[pl.ds(0, NCHUNK * 4)]),
            (src.at[pl.ds(e0, NCHUNK), pl.ds(off, 64)], rwb[sl]),
        )

    def _issue_a(j, sl, rwg):
        for src, dst in _a_copies(j, sl, rwg):
            pltpu.async_copy(src, dst, semA[sl])

    def _wait_a(j, sl, rwg):
        for src, dst in _a_copies(j, sl, rwg):
            pltpu.make_async_copy(src, dst, semA[sl]).wait()

    def _issue_g(sl, hg):
        pltpu.async_copy(hg.at[sidx[sl]], hb[sl], semG[sl])

    def _wait_g(sl, hg):
        pltpu.make_async_copy(hg.at[sidx[sl]], hb[sl], semG[sl]).wait()

    def _copy_ridx(sl):
        for q in range(NCHUNK // 16):
            rs[sl][pl.ds(q * 16, 16)] = ridx[sl][pl.ds(q * 16, 16)]

    def _issue_s(sl):
        pltpu.async_copy(mb[sl], acc.at[rs[sl]], semS[sl], add=True)

    def _wait_s(sl):
        pltpu.make_async_copy(mb[sl], acc.at[rs[sl]], semS[sl]).wait()

    def _compute(sl):
        hbuf, rwbuf, shbuf = hb[sl], rwb[sl], shb[sl]
        msgbuf = mb[sl]

        @pl.loop(0, NCHUNK)
        def _edge(i):
            ha = hbuf[i, 0:16]
            hb_ = hbuf[i, 16:32]
            r0a = rwbuf[i, 0:16]
            r0b = rwbuf[i, 16:32]
            r1a = rwbuf[i, 32:48]
            r1b = rwbuf[i, 48:64]
            sv = shbuf[pl.ds(i * 4, 16)]
            s1 = sv[c1]
            s2 = sv[c2]
            s3 = sv[c3]
            h1a = ha * r1a
            h1b = hb_ * r1b
            msgbuf[i, 0:16] = ha * r0a
            msgbuf[i, 16:32] = hb_ * r0b
            msgbuf[i, 32:48] = h1a * s1
            msgbuf[i, 48:64] = h1b * s1
            msgbuf[i, 64:80] = h1a * s2
            msgbuf[i, 80:96] = h1b * s2
            msgbuf[i, 96:112] = h1a * s3
            msgbuf[i, 112:128] = h1b * s3

    def _pipeline(hg, rwg):
        _issue_a(0, 0, rwg)
        _wait_a(0, 0, rwg)
        _copy_ridx(0)
        _issue_g(0, hg)
        _issue_a(1, 1, rwg)

        @pl.loop(0, NCH // 2)
        def _chunk(jj):
            for half in range(2):
                sl, o = half, 1 - half
                j = jj * 2 + half

                @pl.when(j + 1 < NCH)
                def _():
                    @pl.when(j >= 1)
                    def _():
                        _wait_s(o)

                    _wait_a(j + 1, o, rwg)
                    _copy_ridx(o)
                    _issue_g(o, hg)

                _wait_g(sl, hg)
                _compute(sl)
                _issue_s(sl)

                @pl.when(j + 2 < NCH)
                def _():
                    _issue_a(j + 2, sl, rwg)

        _wait_s(0)
        _wait_s(1)

    for p in range(2):
        @pl.loop(0, NCHUNK)
        def _z(i):
            for jz in range(8):
                mb0[i, 16 * jz:16 * (jz + 1)] = zero16

        for z in range(NACC // (NSUB * NCHUNK)):
            pltpu.sync_copy(mb0, acc.at[pl.ds((s * (NACC // (NSUB * NCHUNK)) + z) * NCHUNK, NCHUNK)])
        plsc.subcore_barrier()

        for cs in range(NCORES):
            g = NCORES * cs + p

            @pl.when(c == cs)
            def _():
                _pipeline(hgs[g], (rwp[p], 64 * cs))

        plsc.subcore_barrier()
        for cs in range(NCORES):
            g = NCORES * cs + p

            @pl.when(c == cs)
            def _():
                pltpu.sync_copy(acc.at[pl.ds(s * NWB, NWB)],
                                aggs[g].at[pl.ds(s * NWB, NWB)])
        plsc.subcore_barrier()


def _message(send_p, recv_p, hgs, rwpair, sh_p):
    mesh = plsc.VectorSubcoreMesh(core_axis_name="c", subcore_axis_name="s")
    return pl.kernel(
        _msg_body,
        out_type=[jax.ShapeDtypeStruct((NACC, C), _F32)] * 4,
        mesh=mesh,
        compiler_params=pltpu.CompilerParams(needs_layout_passes=False,
                                             use_tc_tiling_on_sc=False),
        scratch_types=[
            pltpu.MemorySpace.VMEM_SHARED((NACC, C), _F32),
            pltpu.MemorySpace.VMEM((NCHUNK,), _I32),
            pltpu.MemorySpace.VMEM((NCHUNK,), _I32),
            pltpu.MemorySpace.VMEM((NCHUNK,), _I32),
            pltpu.MemorySpace.VMEM((NCHUNK,), _I32),
            pltpu.MemorySpace.VMEM((NCHUNK,), _I32),
            pltpu.MemorySpace.VMEM((NCHUNK,), _I32),
            pltpu.MemorySpace.VMEM((NCHUNK * 4 + 16,), _F32),
            pltpu.MemorySpace.VMEM((NCHUNK * 4 + 16,), _F32),
            pltpu.MemorySpace.VMEM((NCHUNK, 64), _F32),
            pltpu.MemorySpace.VMEM((NCHUNK, 64), _F32),
            pltpu.MemorySpace.VMEM((NCHUNK, 32), _F32),
            pltpu.MemorySpace.VMEM((NCHUNK, 32), _F32),
            pltpu.MemorySpace.VMEM((NCHUNK, C), _F32),
            pltpu.MemorySpace.VMEM((NCHUNK, C), _F32),
            pltpu.SemaphoreType.DMA,
            pltpu.SemaphoreType.DMA,
            pltpu.SemaphoreType.DMA,
            pltpu.SemaphoreType.DMA,
            pltpu.SemaphoreType.DMA,
            pltpu.SemaphoreType.DMA,
        ],
    )(send_p, recv_p, *hgs, *rwpair, sh_p)


def _edge_body(vec_ref, w1_ref, w2_ref, w3a_ref, w3b_ref,
               rwA0, rwB0, rwA1, rwB1, sh_ref):
    vec = vec_ref[...]
    d2 = jnp.sum(vec * vec, axis=1, keepdims=True) + 1e-12
    r = jnp.sqrt(d2)
    inv_r = 1.0 / r
    unit = vec * inv_r
    sh_ref[...] = jnp.concatenate(
        [jnp.ones((vec.shape[0], 1), _F32), np.sqrt(3.0).astype(np.float32) * unit], axis=1)
    n = (lax.broadcasted_iota(_I32, (vec.shape[0], NB), 1).astype(_F32)
         + 1.0) * (np.pi / RMAX)
    arg = r * n
    u = r * (1.0 / RMAX)
    u5 = u * u * u * u * u
    env = 1.0 - 21.0 * u5 + 35.0 * u5 * u - 15.0 * u5 * u * u
    env = jnp.where(u < 1.0, env, 0.0)
    pref = np.sqrt(2.0 / RMAX).astype(np.float32)
    ef = (pref * jnp.sin(arg)) * (inv_r * env)
    bf = jnp.bfloat16
    t = C2M * _silu(jnp.dot(ef.astype(bf), w1_ref[...],
                            preferred_element_type=_F32))
    t = C2M * _silu(jnp.dot(t.astype(bf), w2_ref[...],
                            preferred_element_type=_F32))
    t16 = t.astype(bf)
    full0 = jnp.dot(t16[:, 0:64], w3a_ref[...], preferred_element_type=_F32)
    full1 = jnp.dot(t16[:, 64:128], w3b_ref[...], preferred_element_type=_F32)
    rwA0[...] = full0[:, 0:128]
    rwB0[...] = full0[:, 128:256]
    rwA1[...] = full1[:, 0:128]
    rwB1[...] = full1[:, 128:256]


def _edge_pass(vec, w1c, w2b, w3a, w3b):
    return pl.pallas_call(
        _edge_body,
        grid=(EP // BE,),
        in_specs=[
            pl.BlockSpec((BE, 3), lambda i: (i, 0)),
            pl.BlockSpec((NB, 2 * 64), lambda i: (0, 0)),
            pl.BlockSpec((2 * 64, 2 * 64), lambda i: (0, 0)),
            pl.BlockSpec((64, 2 * C), lambda i: (0, 0)),
            pl.BlockSpec((64, 2 * C), lambda i: (0, 0)),
        ],
        out_specs=[pl.BlockSpec((BE, 2 * 64), lambda i: (i, 0))] * 4
        + [pl.BlockSpec((BE, 4), lambda i: (i, 0))],
        out_shape=[jax.ShapeDtypeStruct((EP, 2 * 64), _F32)] * 4
        + [jax.ShapeDtypeStruct((EP, 4), _F32)],
    )(vec, w1c, w2b, w3a, w3b)


def _embed_body(na_ref, wemb_ref, e0w_ref, hg0, hg1, hg2, hg3, e_ref):
    na = na_ref[...]
    h = jnp.dot(na, wemb_ref[...], preferred_element_type=_F32)
    for g in range(4):
        (hg0, hg1, hg2, hg3)[g][...] = h[:, 32 * g:32 * (g + 1)]
    e_ref[...] = jnp.dot(na, e0w_ref[...], preferred_element_type=_F32)


def _embed(node_attrs, W_emb, E0_w):
    return pl.pallas_call(
        _embed_body,
        grid=(N // BN,),
        in_specs=[
            pl.BlockSpec((BN, NE), lambda i: (i, 0)),
            pl.BlockSpec((NE, C), lambda i: (0, 0)),
            pl.BlockSpec((NE, 1), lambda i: (0, 0)),
        ],
        out_specs=[pl.BlockSpec((BN, 32), lambda i: (i, 0))] * 4
        + [pl.BlockSpec((BN, 1), lambda i: (i, 0))],
        out_shape=[jax.ShapeDtypeStruct((N, 32), _F32)] * 4
        + [jax.ShapeDtypeStruct((N, 1), _F32)],
    )(node_attrs, W_emb, E0_w.reshape(NE, 1))


def _inv_from_agg(agg_refs):
    pieces = []
    for g in range(4):
        a = agg_refs[g][...]
        a0 = a[:, 0:32] * (1.0 / AVG)
        sq = (a[:, 32:64] ** 2 + a[:, 64:96] ** 2 + a[:, 96:128] ** 2) * (1.0 / (AVG * AVG))
        pieces.append(a0 + sq)
    return jnp.concatenate(pieces, axis=1)


def _node0_body(a0, a1, a2, a3, wm_ref, wro_ref, hg0, hg1, hg2, hg3, e_ref):
    inv = _inv_from_agg((a0, a1, a2, a3))
    h = jnp.dot(inv, wm_ref[...], preferred_element_type=_F32)
    for g in range(4):
        (hg0, hg1, hg2, hg3)[g][...] = h[:, 32 * g:32 * (g + 1)]
    e_ref[...] = jnp.dot(h, wro_ref[...], preferred_element_type=_F32)


def _node0(aggs, Wmix, Wro):
    return pl.pallas_call(
        _node0_body,
        grid=(N // BN,),
        in_specs=[pl.BlockSpec((BN, C), lambda i: (i, 0))] * 4
        + [pl.BlockSpec((C, C), lambda i: (0, 0)),
           pl.BlockSpec((C, 1), lambda i: (0, 0))],
        out_specs=[pl.BlockSpec((BN, 32), lambda i: (i, 0))] * 4
        + [pl.BlockSpec((BN, 1), lambda i: (i, 0))],
        out_shape=[jax.ShapeDtypeStruct((N, 32), _F32)] * 4
        + [jax.ShapeDtypeStruct((N, 1), _F32)],
    )(*aggs, Wmix, Wro)


def _node1_body(a0, a1, a2, a3, wm_ref, wh_ref, wo_ref, e_ref):
    inv = _inv_from_agg((a0, a1, a2, a3))
    h = jnp.dot(inv, wm_ref[...], preferred_element_type=_F32)
    hh = C2M * _silu(jnp.dot(h, wh_ref[...], preferred_element_type=_F32))
    e_ref[...] = jnp.dot(hh, wo_ref[...], preferred_element_type=_F32)


def _node1(aggs, Wmix, Wh, Wo):
    return pl.pallas_call(
        _node1_body,
        grid=(N // BN,),
        in_specs=[pl.BlockSpec((BN, C), lambda i: (i, 0))] * 4
        + [pl.BlockSpec((C, C), lambda i: (0, 0)),
           pl.BlockSpec((C, 16), lambda i: (0, 0)),
           pl.BlockSpec((16, 1), lambda i: (0, 0))],
        out_specs=pl.BlockSpec((BN, 1), lambda i: (i, 0)),
        out_shape=jax.ShapeDtypeStruct((N, 1), _F32),
    )(*aggs, Wmix, Wh, Wo)


def kernel(node_attrs, positions, shifts, W_emb, E0_w,
           rW1_0, rW2_0, rW3_0, Wmix_0, Wro_0,
           rW1_1, rW2_1, rW3_1, Wmix_1, Wh, Wo, edge_index):
    sender = edge_index[0].astype(_I32)
    receiver = edge_index[1].astype(_I32)
    pad = EP - E
    send_p = jnp.concatenate([sender, jnp.zeros((pad,), _I32)])
    recv_g = jnp.concatenate([receiver, jnp.zeros((pad,), _I32)])
    recv_m = jnp.concatenate([receiver, jnp.full((pad,), DUMMY, _I32)])

    perm = np.array([(32 * g + cp) * 2 + path
                     for g in (0, 2, 1, 3) for path in range(2) for cp in range(32)])
    bf = jnp.bfloat16
    w3a = rW3_0[:, perm].astype(bf)
    w3b = rW3_1[:, perm].astype(bf)
    w1c = jnp.concatenate([rW1_0, rW1_1], axis=1).astype(bf)
    w2b = jnp.zeros((128, 128), _F32)
    w2b = w2b.at[0:64, 0:64].set(rW2_0).at[64:128, 64:128].set(rW2_1).astype(bf)

    vec = _geometry(positions.reshape(N * 3), send_p, recv_g).reshape(EP, 3)

    eouts = _edge_pass(vec, w1c, w2b, w3a, w3b)
    rwg0, rwg1, sh_p = eouts[0:2], eouts[2:4], eouts[4].reshape(EP * 4)

    *hgs, e0 = _embed(node_attrs, W_emb, E0_w)
    e = e0[:, 0]

    aggs = _message(send_p, recv_m, hgs, rwg0, sh_p)
    *hgs, ep1 = _node0(aggs, Wmix_0, Wro_0)
    e = e + ep1[:, 0]

    aggs = _message(send_p, recv_m, hgs, rwg1, sh_p)
    e = e + _node1(aggs, Wmix_1, Wh, Wo)[:, 0]
    return e
